# Initial kernel scaffold; baseline (speedup 1.0000x reference)
import jax
import jax.numpy as jnp
from jax.experimental import pallas as pl


def kernel(x, edge_index, W1, b1, g1, be1, W2, b2, g2, be2, Wg1, bg1, Wg2, bg2, Wg3, bg3):
    raise NotImplementedError("write your pallas kernel here")


# speedup vs baseline: 22.5601x; 22.5601x over previous
import functools

import jax
import jax.numpy as jnp
from jax import lax
from jax.experimental import pallas as pl
from jax.experimental.pallas import tpu as pltpu
from jax.experimental.pallas import tpu_sc as plsc

NC = 2
NS = 16
F = 16
BN = 2048

_HIGH = lax.Precision.HIGHEST


def _npad(n):
    return ((n + 1 + BN - 1) // BN) * BN


def _sc_setup(E, N_PAD, trash, C=2000):
    epw = E // (NC * NS)
    niter = epw // C
    rpt = N_PAD // NS
    mesh = plsc.VectorSubcoreMesh(core_axis_name="c", subcore_axis_name="s")

    @functools.partial(
        pl.kernel,
        out_type=(
            jax.ShapeDtypeStruct((NC, N_PAD), jnp.float32),
            jax.ShapeDtypeStruct((E,), jnp.int32),
        ),
        mesh=mesh,
        scratch_types=[
            pltpu.VMEM((C,), jnp.int32),
            pltpu.VMEM((C,), jnp.int32),
            pltpu.VMEM((C,), jnp.float32),
            pltpu.VMEM((C,), jnp.int32),
            pltpu.VMEM((rpt,), jnp.float32),
            pltpu.VMEM_SHARED((N_PAD,), jnp.float32),
        ],
    )
    def setup(row, col, degp, colp, rbuf, cbuf, wbuf, cpbuf, zbuf, dacc):
        c = lax.axis_index("c")
        s = lax.axis_index("s")
        w = s * NC + c

        def zrow(i, _):
            zbuf[pl.ds(i * 16, 16)] = jnp.zeros((16,), jnp.float32)
            return 0

        lax.fori_loop(0, rpt // 16, zrow, 0)
        pltpu.sync_copy(zbuf, dacc.at[pl.ds(s * rpt, rpt)])
        plsc.subcore_barrier()

        def body(i, _):
            base = w * epw + i * C
            pltpu.sync_copy(row.at[pl.ds(base, C)], rbuf)
            pltpu.sync_copy(col.at[pl.ds(base, C)], cbuf)

            def vec(k, _):
                sl = pl.ds(k * 16, 16)
                r = rbuf[sl]
                cc = cbuf[sl]
                m = r == cc
                wbuf[sl] = jnp.where(m, 0.0, 1.0).astype(jnp.float32)
                cpbuf[sl] = jnp.where(m, trash, cc)
                return 0

            lax.fori_loop(0, C // 16, vec, 0)
            pltpu.sync_copy(wbuf, dacc.at[rbuf], add=True)
            pltpu.sync_copy(cpbuf, colp.at[pl.ds(base, C)])
            return 0

        lax.fori_loop(0, niter, body, 0)
        plsc.subcore_barrier()
        pltpu.sync_copy(dacc.at[pl.ds(s * rpt, rpt)],
                        degp.at[c, pl.ds(s * rpt, rpt)])

    return setup


def _sc_layer(E, N_PAD, C=1000):
    ept = E // NS
    niter = ept // C
    rpt = N_PAD // NS
    mesh = plsc.VectorSubcoreMesh(core_axis_name="c", subcore_axis_name="s")

    @functools.partial(
        pl.kernel,
        out_type=(
            jax.ShapeDtypeStruct((N_PAD, F), jnp.float32),
            jax.ShapeDtypeStruct((N_PAD, F), jnp.float32),
        ),
        mesh=mesh,
        scratch_types=[
            pltpu.VMEM((C,), jnp.int32),
            pltpu.VMEM((C,), jnp.int32),
            pltpu.VMEM((C, F), jnp.float32),
            pltpu.VMEM_SHARED((N_PAD, F), jnp.float32),
            pltpu.SemaphoreType.DMA,
        ],
        compiler_params=pltpu.CompilerParams(use_tc_tiling_on_sc=False),
    )
    def layer(row, colp, ulo, uhi, alo, ahi, rbuf, cbuf, gbuf, acc, sem):
        c = lax.axis_index("c")
        s = lax.axis_index("s")

        def zrow(i, _):
            gbuf[i, :] = jnp.zeros((F,), jnp.float32)
            return 0

        lax.fori_loop(0, C, zrow, 0)
        base = s * rpt
        done = 0
        while done < rpt:
            step = min(C, rpt - done)
            pltpu.sync_copy(gbuf.at[pl.ds(0, step)],
                            acc.at[pl.ds(base + done, step)])
            done += step
        plsc.subcore_barrier()

        def run(u_hbm):
            def body(i, _):
                eb = s * ept + i * C
                pltpu.sync_copy(row.at[pl.ds(eb, C)], rbuf)
                pltpu.sync_copy(colp.at[pl.ds(eb, C)], cbuf)
                pltpu.async_copy(u_hbm.at[rbuf], gbuf, sem).wait()
                pltpu.sync_copy(gbuf, acc.at[cbuf], add=True)
                return 0

            lax.fori_loop(0, niter, body, 0)

        @pl.when(c == 0)
        def _():
            run(ulo)

        @pl.when(c == 1)
        def _():
            run(uhi)

        plsc.subcore_barrier()

        @pl.when(c == 0)
        def _():
            pltpu.sync_copy(acc.at[pl.ds(s * rpt, rpt)],
                            alo.at[pl.ds(s * rpt, rpt)])

        @pl.when(c == 1)
        def _():
            pltpu.sync_copy(acc.at[pl.ds(s * rpt, rpt)],
                            ahi.at[pl.ds(s * rpt, rpt)])

    return layer


def _dis_of(degp_blk):
    deg = degp_blk[0, :] + degp_blk[1, :] + 2.0
    return lax.rsqrt(deg)[:, None]


def _row_mask(n):
    rows = pl.program_id(0) * BN + lax.broadcasted_iota(jnp.int32, (BN, 1), 0)
    return rows < n


def _enc1_body(n, x_ref, w_ref, b_ref, u_ref, st_ref):
    u = jnp.dot(x_ref[...], w_ref[...], precision=_HIGH,
                preferred_element_type=jnp.float32) + b_ref[...]
    u_ref[...] = u
    um = jnp.where(_row_mask(n), u, 0.0)
    st = jnp.stack([jnp.sum(um, axis=0), jnp.sum(um * um, axis=0)])

    @pl.when(pl.program_id(0) == 0)
    def _():
        st_ref[...] = st

    @pl.when(pl.program_id(0) > 0)
    def _():
        st_ref[...] += st


def _bn_elu(u, st, g, be, n):
    mean = st[0:1, :] / n
    var = st[1:2, :] / n - mean * mean
    h = (u - mean) * lax.rsqrt(var + 0.001) * g + be
    return jnp.where(h > 0, h, jnp.exp(h) - 1.0)


def _enc2_body(n, u_ref, st_ref, g_ref, be_ref, w_ref, b_ref, v_ref, st2_ref):
    h = _bn_elu(u_ref[...], st_ref[...], g_ref[...], be_ref[...], n)
    v = jnp.dot(h, w_ref[...], precision=_HIGH,
                preferred_element_type=jnp.float32) + b_ref[...]
    v_ref[...] = v
    vm = jnp.where(_row_mask(n), v, 0.0)
    st = jnp.stack([jnp.sum(vm, axis=0), jnp.sum(vm * vm, axis=0)])

    @pl.when(pl.program_id(0) == 0)
    def _():
        st2_ref[...] = st

    @pl.when(pl.program_id(0) > 0)
    def _():
        st2_ref[...] += st


def _mm1_body(n, v_ref, st_ref, g_ref, be_ref, degp_ref, w_ref,
              ulo_ref, uhi_ref):
    h = _bn_elu(v_ref[...], st_ref[...], g_ref[...], be_ref[...], n)
    t = jnp.dot(h, w_ref[...], precision=_HIGH,
                preferred_element_type=jnp.float32)
    u = _dis_of(degp_ref[...]) * t
    ulo_ref[...] = u[:, :F]
    uhi_ref[...] = u[:, F:]


def _mid_body(relu, alo_ref, ahi_ref, ulo_ref, uhi_ref, degp_ref, b_ref,
              w_ref, olo_ref, ohi_ref):
    dis = _dis_of(degp_ref[...])
    agg = jnp.concatenate([alo_ref[...], ahi_ref[...]], axis=1)
    up = jnp.concatenate([ulo_ref[...], uhi_ref[...]], axis=1)
    z = dis * agg + 2.0 * dis * up + b_ref[...]
    if relu:
        z = jnp.maximum(z, 0.0)
    t = jnp.dot(z, w_ref[...], precision=_HIGH,
                preferred_element_type=jnp.float32)
    u = dis * t
    olo_ref[...] = u[:, :F]
    ohi_ref[...] = u[:, F:]


def _fin_body(alo_ref, ahi_ref, ulo_ref, uhi_ref, degp_ref, b_ref, o_ref):
    dis = _dis_of(degp_ref[...])
    agg = jnp.concatenate([alo_ref[...], ahi_ref[...]], axis=1)
    up = jnp.concatenate([ulo_ref[...], uhi_ref[...]], axis=1)
    o_ref[...] = dis * agg + 2.0 * dis * up + b_ref[...]


def _row_spec(cols):
    return pl.BlockSpec((BN, cols), lambda i: (i, 0))


def _full_spec(shape):
    return pl.BlockSpec(shape, lambda i: tuple(0 for _ in shape))


def _degp_spec():
    return pl.BlockSpec((NC, BN), lambda i: (0, i))


def kernel(x, edge_index, W1, b1, g1, be1, W2, b2, g2, be2,
           Wg1, bg1, Wg2, bg2, Wg3, bg3):
    n, d = x.shape
    e = edge_index.shape[1]
    n_pad = _npad(n)
    d1 = W1.shape[1]
    d2 = W2.shape[1]
    grid = (n_pad // BN,)

    b1r, g1r, be1r = b1[None, :], g1[None, :], be1[None, :]
    b2r, g2r, be2r = b2[None, :], g2[None, :], be2[None, :]
    bg1r, bg2r, bg3r = bg1[None, :], bg2[None, :], bg3[None, :]
    xp = jnp.pad(x, ((0, n_pad - n), (0, 0)))
    row = edge_index[0]
    col = edge_index[1]

    degp, colp = _sc_setup(e, n_pad, n)(row, col)

    U, st1 = pl.pallas_call(
        functools.partial(_enc1_body, float(n)),
        grid=grid,
        in_specs=[_row_spec(d), _full_spec((d, d1)), _full_spec((1, d1))],
        out_specs=[_row_spec(d1), _full_spec((2, d1))],
        out_shape=[jax.ShapeDtypeStruct((n_pad, d1), jnp.float32),
                   jax.ShapeDtypeStruct((2, d1), jnp.float32)],
        compiler_params=pltpu.CompilerParams(
            dimension_semantics=("arbitrary",)),
    )(xp, W1, b1r)

    V, st2 = pl.pallas_call(
        functools.partial(_enc2_body, float(n)),
        grid=grid,
        in_specs=[_row_spec(d1), _full_spec((2, d1)),
                  _full_spec((1, d1)), _full_spec((1, d1)),
                  _full_spec((d1, d2)), _full_spec((1, d2))],
        out_specs=[_row_spec(d2), _full_spec((2, d2))],
        out_shape=[jax.ShapeDtypeStruct((n_pad, d2), jnp.float32),
                   jax.ShapeDtypeStruct((2, d2), jnp.float32)],
        compiler_params=pltpu.CompilerParams(
            dimension_semantics=("arbitrary",)),
    )(U, st1, g1r, be1r, W2, b2r)

    ulo, uhi = pl.pallas_call(
        functools.partial(_mm1_body, float(n)),
        grid=grid,
        in_specs=[_row_spec(d2), _full_spec((2, d2)),
                  _full_spec((1, d2)), _full_spec((1, d2)),
                  _degp_spec(), _full_spec((d2, d2))],
        out_specs=[_row_spec(F), _row_spec(F)],
        out_shape=[jax.ShapeDtypeStruct((n_pad, F), jnp.float32),
                   jax.ShapeDtypeStruct((n_pad, F), jnp.float32)],
        compiler_params=pltpu.CompilerParams(
            dimension_semantics=("parallel",)),
    )(V, st2, g2r, be2r, degp, Wg1)

    sc_layer = _sc_layer(e, n_pad)

    def mid(relu, alo, ahi, ulo_, uhi_, bprev, wg):
        return pl.pallas_call(
            functools.partial(_mid_body, relu),
            grid=grid,
            in_specs=[_row_spec(F), _row_spec(F),
                      _row_spec(F), _row_spec(F),
                      _degp_spec(), _full_spec((1, d2)),
                      _full_spec((d2, d2))],
            out_specs=[_row_spec(F), _row_spec(F)],
            out_shape=[jax.ShapeDtypeStruct((n_pad, F), jnp.float32),
                       jax.ShapeDtypeStruct((n_pad, F), jnp.float32)],
            compiler_params=pltpu.CompilerParams(
                dimension_semantics=("parallel",)),
        )(alo, ahi, ulo_, uhi_, degp, bprev, wg)

    alo1, ahi1 = sc_layer(row, colp, ulo, uhi)
    ulo2, uhi2 = mid(True, alo1, ahi1, ulo, uhi, bg1r, Wg2)
    alo2, ahi2 = sc_layer(row, colp, ulo2, uhi2)
    ulo3, uhi3 = mid(False, alo2, ahi2, ulo2, uhi2, bg2r, Wg3)
    alo3, ahi3 = sc_layer(row, colp, ulo3, uhi3)

    out = pl.pallas_call(
        _fin_body,
        grid=grid,
        in_specs=[_row_spec(F), _row_spec(F),
                  _row_spec(F), _row_spec(F),
                  _degp_spec(), _full_spec((1, d2))],
        out_specs=_row_spec(d2),
        out_shape=jax.ShapeDtypeStruct((n_pad, d2), jnp.float32),
        compiler_params=pltpu.CompilerParams(
            dimension_semantics=("parallel",)),
    )(alo3, ahi3, ulo3, uhi3, degp, bg3r)

    return out[:n]


# speedup vs baseline: 28.5138x; 1.2639x over previous
import functools

import jax
import jax.numpy as jnp
from jax import lax
from jax.experimental import pallas as pl
from jax.experimental.pallas import tpu as pltpu
from jax.experimental.pallas import tpu_sc as plsc

NC = 2
NS = 16
F = 16
BN = 2048

_HIGH = lax.Precision.HIGHEST


def _npad(n):
    return ((n + 1 + BN - 1) // BN) * BN


def _sc_setup(E, N_PAD, trash, C=2000):
    epw = E // (NC * NS)
    niter = epw // C
    rpt = N_PAD // NS
    mesh = plsc.VectorSubcoreMesh(core_axis_name="c", subcore_axis_name="s")

    @functools.partial(
        pl.kernel,
        out_type=(
            jax.ShapeDtypeStruct((NC, N_PAD), jnp.float32),
            jax.ShapeDtypeStruct((E,), jnp.int32),
        ),
        mesh=mesh,
        scratch_types=[
            pltpu.VMEM((C,), jnp.int32),
            pltpu.VMEM((C,), jnp.int32),
            pltpu.VMEM((C,), jnp.float32),
            pltpu.VMEM((C,), jnp.int32),
            pltpu.VMEM((rpt,), jnp.float32),
            pltpu.VMEM_SHARED((N_PAD,), jnp.float32),
        ],
    )
    def setup(row, col, degp, colp, rbuf, cbuf, wbuf, cpbuf, zbuf, dacc):
        c = lax.axis_index("c")
        s = lax.axis_index("s")
        w = s * NC + c

        def zrow(i, _):
            zbuf[pl.ds(i * 16, 16)] = jnp.zeros((16,), jnp.float32)
            return 0

        lax.fori_loop(0, rpt // 16, zrow, 0)
        pltpu.sync_copy(zbuf, dacc.at[pl.ds(s * rpt, rpt)])
        plsc.subcore_barrier()

        def body(i, _):
            base = w * epw + i * C
            pltpu.sync_copy(row.at[pl.ds(base, C)], rbuf)
            pltpu.sync_copy(col.at[pl.ds(base, C)], cbuf)

            def vec(k, _):
                sl = pl.ds(k * 16, 16)
                r = rbuf[sl]
                cc = cbuf[sl]
                m = r == cc
                wbuf[sl] = jnp.where(m, 0.0, 1.0).astype(jnp.float32)
                cpbuf[sl] = jnp.where(m, trash, cc)
                return 0

            lax.fori_loop(0, C // 16, vec, 0)
            pltpu.sync_copy(wbuf, dacc.at[rbuf], add=True)
            pltpu.sync_copy(cpbuf, colp.at[pl.ds(base, C)])
            return 0

        lax.fori_loop(0, niter, body, 0)
        plsc.subcore_barrier()
        pltpu.sync_copy(dacc.at[pl.ds(s * rpt, rpt)],
                        degp.at[c, pl.ds(s * rpt, rpt)])

    return setup


def _sc_layer(E, N_PAD, C=800):
    ept = E // NS
    niter = ept // C
    npair = (niter - 1) // 2
    assert niter == 2 * npair + 1
    rpt = N_PAD // NS
    mesh = plsc.VectorSubcoreMesh(core_axis_name="c", subcore_axis_name="s")

    @functools.partial(
        pl.kernel,
        out_type=(
            jax.ShapeDtypeStruct((N_PAD, F), jnp.float32),
            jax.ShapeDtypeStruct((N_PAD, F), jnp.float32),
        ),
        mesh=mesh,
        scratch_types=[
            pltpu.VMEM((C,), jnp.int32),
            pltpu.VMEM((C,), jnp.int32),
            pltpu.VMEM((C,), jnp.int32),
            pltpu.VMEM((C,), jnp.int32),
            pltpu.VMEM((C, F), jnp.float32),
            pltpu.VMEM((C, F), jnp.float32),
            pltpu.VMEM_SHARED((N_PAD, F), jnp.float32),
            pltpu.SemaphoreType.DMA,
            pltpu.SemaphoreType.DMA,
            pltpu.SemaphoreType.DMA,
            pltpu.SemaphoreType.DMA,
        ],
        compiler_params=pltpu.CompilerParams(use_tc_tiling_on_sc=False),
    )
    def layer(row, colp, ulo, uhi, alo, ahi,
              rbA, cbA, rbB, cbB, gbA, gbB, acc, siA, siB, sgA, sgB):
        c = lax.axis_index("c")
        s = lax.axis_index("s")

        def zrow(i, _):
            gbA[i, :] = jnp.zeros((F,), jnp.float32)
            return 0

        lax.fori_loop(0, C, zrow, 0)
        base = s * rpt
        done = 0
        while done < rpt:
            step = min(C, rpt - done)
            pltpu.sync_copy(gbA.at[pl.ds(0, step)],
                            acc.at[pl.ds(base + done, step)])
            done += step
        plsc.subcore_barrier()

        def run(u_hbm):
            tb = s * ept

            def issue_idx(k, rb, cb, sem):
                pltpu.async_copy(row.at[pl.ds(tb + k * C, C)], rb, sem)
                pltpu.async_copy(colp.at[pl.ds(tb + k * C, C)], cb, sem)

            def wait_idx(rb, cb, sem):
                pltpu.make_async_copy(row.at[pl.ds(tb, C)], rb, sem).wait()
                pltpu.make_async_copy(colp.at[pl.ds(tb, C)], cb, sem).wait()

            def wait_gather(rb, gb, sem):
                pltpu.make_async_copy(u_hbm.at[rb], gb, sem).wait()

            issue_idx(0, rbA, cbA, siA)
            issue_idx(1, rbB, cbB, siB)
            wait_idx(rbA, cbA, siA)
            pltpu.async_copy(u_hbm.at[rbA], gbA, sgA)

            def pair(j, _):
                wait_idx(rbB, cbB, siB)
                wait_gather(rbA, gbA, sgA)
                pltpu.async_copy(u_hbm.at[rbB], gbB, sgB)
                pltpu.sync_copy(gbA, acc.at[cbA], add=True)
                issue_idx(2 * j + 2, rbA, cbA, siA)
                wait_gather(rbB, gbB, sgB)
                wait_idx(rbA, cbA, siA)
                pltpu.async_copy(u_hbm.at[rbA], gbA, sgA)
                pltpu.sync_copy(gbB, acc.at[cbB], add=True)

                @pl.when(j < npair - 1)
                def _():
                    issue_idx(2 * j + 3, rbB, cbB, siB)

                return 0

            lax.fori_loop(0, npair, pair, 0)
            wait_gather(rbA, gbA, sgA)
            pltpu.sync_copy(gbA, acc.at[cbA], add=True)

        @pl.when(c == 0)
        def _():
            run(ulo)

        @pl.when(c == 1)
        def _():
            run(uhi)

        plsc.subcore_barrier()

        @pl.when(c == 0)
        def _():
            pltpu.sync_copy(acc.at[pl.ds(s * rpt, rpt)],
                            alo.at[pl.ds(s * rpt, rpt)])

        @pl.when(c == 1)
        def _():
            pltpu.sync_copy(acc.at[pl.ds(s * rpt, rpt)],
                            ahi.at[pl.ds(s * rpt, rpt)])

    return layer


def _dis_of(degp_blk):
    deg = degp_blk[0, :] + degp_blk[1, :] + 2.0
    return lax.rsqrt(deg)[:, None]


def _row_mask(n):
    rows = pl.program_id(0) * BN + lax.broadcasted_iota(jnp.int32, (BN, 1), 0)
    return rows < n


def _enc1_body(n, x_ref, w_ref, b_ref, u_ref, st_ref):
    u = jnp.dot(x_ref[...], w_ref[...], precision=_HIGH,
                preferred_element_type=jnp.float32) + b_ref[...]
    u_ref[...] = u
    um = jnp.where(_row_mask(n), u, 0.0)
    st = jnp.stack([jnp.sum(um, axis=0), jnp.sum(um * um, axis=0)])

    @pl.when(pl.program_id(0) == 0)
    def _():
        st_ref[...] = st

    @pl.when(pl.program_id(0) > 0)
    def _():
        st_ref[...] += st


def _bn_elu(u, st, g, be, n):
    mean = st[0:1, :] / n
    var = st[1:2, :] / n - mean * mean
    h = (u - mean) * lax.rsqrt(var + 0.001) * g + be
    return jnp.where(h > 0, h, jnp.exp(h) - 1.0)


def _enc2_body(n, u_ref, st_ref, g_ref, be_ref, w_ref, b_ref, v_ref, st2_ref):
    h = _bn_elu(u_ref[...], st_ref[...], g_ref[...], be_ref[...], n)
    v = jnp.dot(h, w_ref[...], precision=_HIGH,
                preferred_element_type=jnp.float32) + b_ref[...]
    v_ref[...] = v
    vm = jnp.where(_row_mask(n), v, 0.0)
    st = jnp.stack([jnp.sum(vm, axis=0), jnp.sum(vm * vm, axis=0)])

    @pl.when(pl.program_id(0) == 0)
    def _():
        st2_ref[...] = st

    @pl.when(pl.program_id(0) > 0)
    def _():
        st2_ref[...] += st


def _mm1_body(n, v_ref, st_ref, g_ref, be_ref, degp_ref, w_ref,
              ulo_ref, uhi_ref):
    h = _bn_elu(v_ref[...], st_ref[...], g_ref[...], be_ref[...], n)
    t = jnp.dot(h, w_ref[...], precision=_HIGH,
                preferred_element_type=jnp.float32)
    u = _dis_of(degp_ref[...]) * t
    ulo_ref[...] = u[:, :F]
    uhi_ref[...] = u[:, F:]


def _mid_body(relu, alo_ref, ahi_ref, ulo_ref, uhi_ref, degp_ref, b_ref,
              w_ref, olo_ref, ohi_ref):
    dis = _dis_of(degp_ref[...])
    agg = jnp.concatenate([alo_ref[...], ahi_ref[...]], axis=1)
    up = jnp.concatenate([ulo_ref[...], uhi_ref[...]], axis=1)
    z = dis * agg + 2.0 * dis * up + b_ref[...]
    if relu:
        z = jnp.maximum(z, 0.0)
    t = jnp.dot(z, w_ref[...], precision=_HIGH,
                preferred_element_type=jnp.float32)
    u = dis * t
    olo_ref[...] = u[:, :F]
    ohi_ref[...] = u[:, F:]


def _fin_body(alo_ref, ahi_ref, ulo_ref, uhi_ref, degp_ref, b_ref, o_ref):
    dis = _dis_of(degp_ref[...])
    agg = jnp.concatenate([alo_ref[...], ahi_ref[...]], axis=1)
    up = jnp.concatenate([ulo_ref[...], uhi_ref[...]], axis=1)
    o_ref[...] = dis * agg + 2.0 * dis * up + b_ref[...]


def _row_spec(cols):
    return pl.BlockSpec((BN, cols), lambda i: (i, 0))


def _full_spec(shape):
    return pl.BlockSpec(shape, lambda i: tuple(0 for _ in shape))


def _degp_spec():
    return pl.BlockSpec((NC, BN), lambda i: (0, i))


def kernel(x, edge_index, W1, b1, g1, be1, W2, b2, g2, be2,
           Wg1, bg1, Wg2, bg2, Wg3, bg3):
    n, d = x.shape
    e = edge_index.shape[1]
    n_pad = _npad(n)
    d1 = W1.shape[1]
    d2 = W2.shape[1]
    grid = (n_pad // BN,)

    b1r, g1r, be1r = b1[None, :], g1[None, :], be1[None, :]
    b2r, g2r, be2r = b2[None, :], g2[None, :], be2[None, :]
    bg1r, bg2r, bg3r = bg1[None, :], bg2[None, :], bg3[None, :]
    xp = jnp.pad(x, ((0, n_pad - n), (0, 0)))
    row = edge_index[0]
    col = edge_index[1]

    degp, colp = _sc_setup(e, n_pad, n)(row, col)

    U, st1 = pl.pallas_call(
        functools.partial(_enc1_body, float(n)),
        grid=grid,
        in_specs=[_row_spec(d), _full_spec((d, d1)), _full_spec((1, d1))],
        out_specs=[_row_spec(d1), _full_spec((2, d1))],
        out_shape=[jax.ShapeDtypeStruct((n_pad, d1), jnp.float32),
                   jax.ShapeDtypeStruct((2, d1), jnp.float32)],
        compiler_params=pltpu.CompilerParams(
            dimension_semantics=("arbitrary",)),
    )(xp, W1, b1r)

    V, st2 = pl.pallas_call(
        functools.partial(_enc2_body, float(n)),
        grid=grid,
        in_specs=[_row_spec(d1), _full_spec((2, d1)),
                  _full_spec((1, d1)), _full_spec((1, d1)),
                  _full_spec((d1, d2)), _full_spec((1, d2))],
        out_specs=[_row_spec(d2), _full_spec((2, d2))],
        out_shape=[jax.ShapeDtypeStruct((n_pad, d2), jnp.float32),
                   jax.ShapeDtypeStruct((2, d2), jnp.float32)],
        compiler_params=pltpu.CompilerParams(
            dimension_semantics=("arbitrary",)),
    )(U, st1, g1r, be1r, W2, b2r)

    ulo, uhi = pl.pallas_call(
        functools.partial(_mm1_body, float(n)),
        grid=grid,
        in_specs=[_row_spec(d2), _full_spec((2, d2)),
                  _full_spec((1, d2)), _full_spec((1, d2)),
                  _degp_spec(), _full_spec((d2, d2))],
        out_specs=[_row_spec(F), _row_spec(F)],
        out_shape=[jax.ShapeDtypeStruct((n_pad, F), jnp.float32),
                   jax.ShapeDtypeStruct((n_pad, F), jnp.float32)],
        compiler_params=pltpu.CompilerParams(
            dimension_semantics=("parallel",)),
    )(V, st2, g2r, be2r, degp, Wg1)

    sc_layer = _sc_layer(e, n_pad)

    def mid(relu, alo, ahi, ulo_, uhi_, bprev, wg):
        return pl.pallas_call(
            functools.partial(_mid_body, relu),
            grid=grid,
            in_specs=[_row_spec(F), _row_spec(F),
                      _row_spec(F), _row_spec(F),
                      _degp_spec(), _full_spec((1, d2)),
                      _full_spec((d2, d2))],
            out_specs=[_row_spec(F), _row_spec(F)],
            out_shape=[jax.ShapeDtypeStruct((n_pad, F), jnp.float32),
                       jax.ShapeDtypeStruct((n_pad, F), jnp.float32)],
            compiler_params=pltpu.CompilerParams(
                dimension_semantics=("parallel",)),
        )(alo, ahi, ulo_, uhi_, degp, bprev, wg)

    alo1, ahi1 = sc_layer(row, colp, ulo, uhi)
    ulo2, uhi2 = mid(True, alo1, ahi1, ulo, uhi, bg1r, Wg2)
    alo2, ahi2 = sc_layer(row, colp, ulo2, uhi2)
    ulo3, uhi3 = mid(False, alo2, ahi2, ulo2, uhi2, bg2r, Wg3)
    alo3, ahi3 = sc_layer(row, colp, ulo3, uhi3)

    out = pl.pallas_call(
        _fin_body,
        grid=grid,
        in_specs=[_row_spec(F), _row_spec(F),
                  _row_spec(F), _row_spec(F),
                  _degp_spec(), _full_spec((1, d2))],
        out_specs=_row_spec(d2),
        out_shape=jax.ShapeDtypeStruct((n_pad, d2), jnp.float32),
        compiler_params=pltpu.CompilerParams(
            dimension_semantics=("parallel",)),
    )(alo3, ahi3, ulo3, uhi3, degp, bg3r)

    return out[:n]


# speedup vs baseline: 29.3941x; 1.0309x over previous
import functools

import jax
import jax.numpy as jnp
from jax import lax
from jax.experimental import pallas as pl
from jax.experimental.pallas import tpu as pltpu
from jax.experimental.pallas import tpu_sc as plsc

NC = 2
NS = 16
F = 16
BN = 4096

_HIGH = lax.Precision.HIGHEST


def _npad(n):
    return ((n + 1 + BN - 1) // BN) * BN


def _sc_setup(E, N_PAD, trash, C=2000):
    epw = E // (NC * NS)
    niter = epw // C
    rpt = N_PAD // NS
    mesh = plsc.VectorSubcoreMesh(core_axis_name="c", subcore_axis_name="s")

    @functools.partial(
        pl.kernel,
        out_type=(
            jax.ShapeDtypeStruct((NC, N_PAD), jnp.float32),
            jax.ShapeDtypeStruct((E,), jnp.int32),
        ),
        mesh=mesh,
        scratch_types=[
            pltpu.VMEM((C,), jnp.int32),
            pltpu.VMEM((C,), jnp.int32),
            pltpu.VMEM((C,), jnp.float32),
            pltpu.VMEM((C,), jnp.int32),
            pltpu.VMEM((rpt,), jnp.float32),
            pltpu.VMEM_SHARED((N_PAD,), jnp.float32),
        ],
    )
    def setup(row, col, degp, colp, rbuf, cbuf, wbuf, cpbuf, zbuf, dacc):
        c = lax.axis_index("c")
        s = lax.axis_index("s")
        w = s * NC + c

        def zrow(i, _):
            zbuf[pl.ds(i * 16, 16)] = jnp.zeros((16,), jnp.float32)
            return 0

        lax.fori_loop(0, rpt // 16, zrow, 0)
        pltpu.sync_copy(zbuf, dacc.at[pl.ds(s * rpt, rpt)])
        plsc.subcore_barrier()

        def body(i, _):
            base = w * epw + i * C
            pltpu.sync_copy(row.at[pl.ds(base, C)], rbuf)
            pltpu.sync_copy(col.at[pl.ds(base, C)], cbuf)

            def vec(k, _):
                sl = pl.ds(k * 16, 16)
                r = rbuf[sl]
                cc = cbuf[sl]
                m = r == cc
                wbuf[sl] = jnp.where(m, 0.0, 1.0).astype(jnp.float32)
                cpbuf[sl] = jnp.where(m, trash, cc)
                return 0

            lax.fori_loop(0, C // 16, vec, 0)
            pltpu.sync_copy(wbuf, dacc.at[rbuf], add=True)
            pltpu.sync_copy(cpbuf, colp.at[pl.ds(base, C)])
            return 0

        lax.fori_loop(0, niter, body, 0)
        plsc.subcore_barrier()
        pltpu.sync_copy(dacc.at[pl.ds(s * rpt, rpt)],
                        degp.at[c, pl.ds(s * rpt, rpt)])

    return setup


def _sc_layer(E, n, N_PAD, C=800):
    ept = E // NS
    niter = ept // C
    npair = (niter - 1) // 2
    assert niter == 2 * npair + 1
    n_acc = ((n + 1 + 127) // 128) * 128
    rpt = n_acc // NS
    mesh = plsc.VectorSubcoreMesh(core_axis_name="c", subcore_axis_name="s")

    @functools.partial(
        pl.kernel,
        out_type=(
            jax.ShapeDtypeStruct((N_PAD, F), jnp.float32),
            jax.ShapeDtypeStruct((N_PAD, F), jnp.float32),
        ),
        mesh=mesh,
        scratch_types=[
            pltpu.VMEM((C,), jnp.int32),
            pltpu.VMEM((C,), jnp.int32),
            pltpu.VMEM((C,), jnp.int32),
            pltpu.VMEM((C,), jnp.int32),
            pltpu.VMEM((C, F), jnp.float32),
            pltpu.VMEM((C, F), jnp.float32),
            pltpu.VMEM_SHARED((n_acc, F), jnp.float32),
            pltpu.SemaphoreType.DMA,
            pltpu.SemaphoreType.DMA,
            pltpu.SemaphoreType.DMA,
            pltpu.SemaphoreType.DMA,
        ],
        compiler_params=pltpu.CompilerParams(use_tc_tiling_on_sc=False),
    )
    def layer(row, colp, ulo, uhi, alo, ahi,
              rbA, cbA, rbB, cbB, gbA, gbB, acc, siA, siB, sgA, sgB):
        c = lax.axis_index("c")
        s = lax.axis_index("s")

        def zrow(i, _):
            gbA[i, :] = jnp.zeros((F,), jnp.float32)
            return 0

        lax.fori_loop(0, C, zrow, 0)
        base = s * rpt
        done = 0
        while done < rpt:
            step = min(C, rpt - done)
            pltpu.sync_copy(gbA.at[pl.ds(0, step)],
                            acc.at[pl.ds(base + done, step)])
            done += step
        plsc.subcore_barrier()

        def run(u_hbm):
            tb = s * ept

            def issue_idx(k, rb, cb, sem):
                pltpu.async_copy(row.at[pl.ds(tb + k * C, C)], rb, sem)
                pltpu.async_copy(colp.at[pl.ds(tb + k * C, C)], cb, sem)

            def wait_idx(rb, cb, sem):
                pltpu.make_async_copy(row.at[pl.ds(tb, C)], rb, sem).wait()
                pltpu.make_async_copy(colp.at[pl.ds(tb, C)], cb, sem).wait()

            def wait_gather(rb, gb, sem):
                pltpu.make_async_copy(u_hbm.at[rb], gb, sem).wait()

            issue_idx(0, rbA, cbA, siA)
            issue_idx(1, rbB, cbB, siB)
            wait_idx(rbA, cbA, siA)
            pltpu.async_copy(u_hbm.at[rbA], gbA, sgA)

            def pair(j, _):
                wait_idx(rbB, cbB, siB)
                wait_gather(rbA, gbA, sgA)
                pltpu.async_copy(u_hbm.at[rbB], gbB, sgB)
                pltpu.sync_copy(gbA, acc.at[cbA], add=True)
                issue_idx(2 * j + 2, rbA, cbA, siA)
                wait_gather(rbB, gbB, sgB)
                wait_idx(rbA, cbA, siA)
                pltpu.async_copy(u_hbm.at[rbA], gbA, sgA)
                pltpu.sync_copy(gbB, acc.at[cbB], add=True)

                @pl.when(j < npair - 1)
                def _():
                    issue_idx(2 * j + 3, rbB, cbB, siB)

                return 0

            lax.fori_loop(0, npair, pair, 0)
            wait_gather(rbA, gbA, sgA)
            pltpu.sync_copy(gbA, acc.at[cbA], add=True)

        @pl.when(c == 0)
        def _():
            run(ulo)

        @pl.when(c == 1)
        def _():
            run(uhi)

        plsc.subcore_barrier()

        @pl.when(c == 0)
        def _():
            pltpu.sync_copy(acc.at[pl.ds(s * rpt, rpt)],
                            alo.at[pl.ds(s * rpt, rpt)])

        @pl.when(c == 1)
        def _():
            pltpu.sync_copy(acc.at[pl.ds(s * rpt, rpt)],
                            ahi.at[pl.ds(s * rpt, rpt)])

    return layer


def _dis_of(degp_blk):
    deg = degp_blk[0, :] + degp_blk[1, :] + 2.0
    return lax.rsqrt(deg)[:, None]


def _row_mask(n):
    rows = pl.program_id(0) * BN + lax.broadcasted_iota(jnp.int32, (BN, 1), 0)
    return rows < n


def _enc1_body(n, x_ref, w_ref, b_ref, u_ref, st_ref):
    u = jnp.dot(x_ref[...], w_ref[...], precision=_HIGH,
                preferred_element_type=jnp.float32) + b_ref[...]
    u_ref[...] = u
    um = jnp.where(_row_mask(n), u, 0.0)
    st = jnp.stack([jnp.sum(um, axis=0), jnp.sum(um * um, axis=0)])

    @pl.when(pl.program_id(0) == 0)
    def _():
        st_ref[...] = st

    @pl.when(pl.program_id(0) > 0)
    def _():
        st_ref[...] += st


def _bn_elu(u, st, g, be, n):
    mean = st[0:1, :] / n
    var = st[1:2, :] / n - mean * mean
    h = (u - mean) * lax.rsqrt(var + 0.001) * g + be
    return jnp.where(h > 0, h, jnp.exp(h) - 1.0)


def _enc2_body(n, u_ref, st_ref, g_ref, be_ref, w_ref, b_ref, v_ref, st2_ref):
    h = _bn_elu(u_ref[...], st_ref[...], g_ref[...], be_ref[...], n)
    v = jnp.dot(h, w_ref[...], precision=_HIGH,
                preferred_element_type=jnp.float32) + b_ref[...]
    v_ref[...] = v
    vm = jnp.where(_row_mask(n), v, 0.0)
    st = jnp.stack([jnp.sum(vm, axis=0), jnp.sum(vm * vm, axis=0)])

    @pl.when(pl.program_id(0) == 0)
    def _():
        st2_ref[...] = st

    @pl.when(pl.program_id(0) > 0)
    def _():
        st2_ref[...] += st


def _mm1_body(n, v_ref, st_ref, g_ref, be_ref, degp_ref, w_ref,
              ulo_ref, uhi_ref):
    h = _bn_elu(v_ref[...], st_ref[...], g_ref[...], be_ref[...], n)
    t = jnp.dot(h, w_ref[...], precision=_HIGH,
                preferred_element_type=jnp.float32)
    u = _dis_of(degp_ref[...]) * t
    ulo_ref[...] = u[:, :F]
    uhi_ref[...] = u[:, F:]


def _mid_body(relu, alo_ref, ahi_ref, ulo_ref, uhi_ref, degp_ref, b_ref,
              w_ref, olo_ref, ohi_ref):
    dis = _dis_of(degp_ref[...])
    agg = jnp.concatenate([alo_ref[...], ahi_ref[...]], axis=1)
    up = jnp.concatenate([ulo_ref[...], uhi_ref[...]], axis=1)
    z = dis * agg + 2.0 * dis * up + b_ref[...]
    if relu:
        z = jnp.maximum(z, 0.0)
    t = jnp.dot(z, w_ref[...], precision=_HIGH,
                preferred_element_type=jnp.float32)
    u = dis * t
    olo_ref[...] = u[:, :F]
    ohi_ref[...] = u[:, F:]


def _fin_body(alo_ref, ahi_ref, ulo_ref, uhi_ref, degp_ref, b_ref, o_ref):
    dis = _dis_of(degp_ref[...])
    agg = jnp.concatenate([alo_ref[...], ahi_ref[...]], axis=1)
    up = jnp.concatenate([ulo_ref[...], uhi_ref[...]], axis=1)
    o_ref[...] = dis * agg + 2.0 * dis * up + b_ref[...]


def _row_spec(cols):
    return pl.BlockSpec((BN, cols), lambda i: (i, 0))


def _full_spec(shape):
    return pl.BlockSpec(shape, lambda i: tuple(0 for _ in shape))


def _degp_spec():
    return pl.BlockSpec((NC, BN), lambda i: (0, i))


def kernel(x, edge_index, W1, b1, g1, be1, W2, b2, g2, be2,
           Wg1, bg1, Wg2, bg2, Wg3, bg3):
    n, d = x.shape
    e = edge_index.shape[1]
    n_pad = _npad(n)
    d1 = W1.shape[1]
    d2 = W2.shape[1]
    grid = (n_pad // BN,)

    b1r, g1r, be1r = b1[None, :], g1[None, :], be1[None, :]
    b2r, g2r, be2r = b2[None, :], g2[None, :], be2[None, :]
    bg1r, bg2r, bg3r = bg1[None, :], bg2[None, :], bg3[None, :]
    xp = jnp.pad(x, ((0, n_pad - n), (0, 0)))
    row = edge_index[0]
    col = edge_index[1]

    degp, colp = _sc_setup(e, n_pad, n)(row, col)

    U, st1 = pl.pallas_call(
        functools.partial(_enc1_body, float(n)),
        grid=grid,
        in_specs=[_row_spec(d), _full_spec((d, d1)), _full_spec((1, d1))],
        out_specs=[_row_spec(d1), _full_spec((2, d1))],
        out_shape=[jax.ShapeDtypeStruct((n_pad, d1), jnp.float32),
                   jax.ShapeDtypeStruct((2, d1), jnp.float32)],
        compiler_params=pltpu.CompilerParams(
            dimension_semantics=("arbitrary",)),
    )(xp, W1, b1r)

    V, st2 = pl.pallas_call(
        functools.partial(_enc2_body, float(n)),
        grid=grid,
        in_specs=[_row_spec(d1), _full_spec((2, d1)),
                  _full_spec((1, d1)), _full_spec((1, d1)),
                  _full_spec((d1, d2)), _full_spec((1, d2))],
        out_specs=[_row_spec(d2), _full_spec((2, d2))],
        out_shape=[jax.ShapeDtypeStruct((n_pad, d2), jnp.float32),
                   jax.ShapeDtypeStruct((2, d2), jnp.float32)],
        compiler_params=pltpu.CompilerParams(
            dimension_semantics=("arbitrary",)),
    )(U, st1, g1r, be1r, W2, b2r)

    ulo, uhi = pl.pallas_call(
        functools.partial(_mm1_body, float(n)),
        grid=grid,
        in_specs=[_row_spec(d2), _full_spec((2, d2)),
                  _full_spec((1, d2)), _full_spec((1, d2)),
                  _degp_spec(), _full_spec((d2, d2))],
        out_specs=[_row_spec(F), _row_spec(F)],
        out_shape=[jax.ShapeDtypeStruct((n_pad, F), jnp.float32),
                   jax.ShapeDtypeStruct((n_pad, F), jnp.float32)],
        compiler_params=pltpu.CompilerParams(
            dimension_semantics=("parallel",)),
    )(V, st2, g2r, be2r, degp, Wg1)

    sc_layer = _sc_layer(e, n, n_pad)

    def mid(relu, alo, ahi, ulo_, uhi_, bprev, wg):
        return pl.pallas_call(
            functools.partial(_mid_body, relu),
            grid=grid,
            in_specs=[_row_spec(F), _row_spec(F),
                      _row_spec(F), _row_spec(F),
                      _degp_spec(), _full_spec((1, d2)),
                      _full_spec((d2, d2))],
            out_specs=[_row_spec(F), _row_spec(F)],
            out_shape=[jax.ShapeDtypeStruct((n_pad, F), jnp.float32),
                       jax.ShapeDtypeStruct((n_pad, F), jnp.float32)],
            compiler_params=pltpu.CompilerParams(
                dimension_semantics=("parallel",)),
        )(alo, ahi, ulo_, uhi_, degp, bprev, wg)

    alo1, ahi1 = sc_layer(row, colp, ulo, uhi)
    ulo2, uhi2 = mid(True, alo1, ahi1, ulo, uhi, bg1r, Wg2)
    alo2, ahi2 = sc_layer(row, colp, ulo2, uhi2)
    ulo3, uhi3 = mid(False, alo2, ahi2, ulo2, uhi2, bg2r, Wg3)
    alo3, ahi3 = sc_layer(row, colp, ulo3, uhi3)

    out = pl.pallas_call(
        _fin_body,
        grid=grid,
        in_specs=[_row_spec(F), _row_spec(F),
                  _row_spec(F), _row_spec(F),
                  _degp_spec(), _full_spec((1, d2))],
        out_specs=_row_spec(d2),
        out_shape=jax.ShapeDtypeStruct((n_pad, d2), jnp.float32),
        compiler_params=pltpu.CompilerParams(
            dimension_semantics=("parallel",)),
    )(alo3, ahi3, ulo3, uhi3, degp, bg3r)

    return out[:n]


# speedup vs baseline: 40.8059x; 1.3882x over previous
import functools

import jax
import jax.numpy as jnp
from jax import lax
from jax.experimental import pallas as pl
from jax.experimental.pallas import tpu as pltpu
from jax.experimental.pallas import tpu_sc as plsc

NC = 2
NS = 16
F = 16
BN = 4096

_HIGH = lax.Precision.HIGHEST


def _npad(n):
    return ((n + 1 + BN - 1) // BN) * BN


def _sc_setup(E, N_PAD, trash, C=2000):
    epw = E // (NC * NS)
    niter = epw // C
    rpt = N_PAD // NS
    mesh = plsc.VectorSubcoreMesh(core_axis_name="c", subcore_axis_name="s")

    RC = 800

    @functools.partial(
        pl.kernel,
        out_type=(
            jax.ShapeDtypeStruct((NC, N_PAD), jnp.float32),
            jax.ShapeDtypeStruct((E,), jnp.int32),
            jax.ShapeDtypeStruct((N_PAD * F,), jnp.float32),
            jax.ShapeDtypeStruct((N_PAD * F,), jnp.float32),
        ),
        mesh=mesh,
        scratch_types=[
            pltpu.VMEM((C,), jnp.int32),
            pltpu.VMEM((C,), jnp.int32),
            pltpu.VMEM((C,), jnp.float32),
            pltpu.VMEM((C,), jnp.int32),
            pltpu.VMEM((rpt,), jnp.float32),
            pltpu.VMEM((RC,), jnp.float32),
            pltpu.VMEM((RC * F,), jnp.float32),
            pltpu.VMEM_SHARED((N_PAD,), jnp.float32),
        ],
        compiler_params=pltpu.CompilerParams(needs_layout_passes=False),
    )
    def setup(row, col, degp, colp, d0rep, d1rep,
              rbuf, cbuf, wbuf, cpbuf, zbuf, dbuf, repbuf, dacc):
        c = lax.axis_index("c")
        s = lax.axis_index("s")
        w = s * NC + c

        def zrow(i, _):
            zbuf[pl.ds(i * 16, 16)] = jnp.zeros((16,), jnp.float32)
            return 0

        lax.fori_loop(0, rpt // 16, zrow, 0)
        pltpu.sync_copy(zbuf, dacc.at[pl.ds(s * rpt, rpt)])
        plsc.subcore_barrier()

        def body(i, _):
            base = w * epw + i * C
            pltpu.sync_copy(row.at[pl.ds(base, C)], rbuf)
            pltpu.sync_copy(col.at[pl.ds(base, C)], cbuf)

            def vec(k, _):
                sl = pl.ds(k * 16, 16)
                r = rbuf[sl]
                cc = cbuf[sl]
                m = r == cc
                wbuf[sl] = jnp.where(m, 0.0, 1.0).astype(jnp.float32)
                cpbuf[sl] = jnp.where(m, trash, cc)
                return 0

            lax.fori_loop(0, C // 16, vec, 0)
            pltpu.sync_copy(wbuf, dacc.at[rbuf], add=True)
            pltpu.sync_copy(cpbuf, colp.at[pl.ds(base, C)])
            return 0

        lax.fori_loop(0, niter, body, 0)
        plsc.subcore_barrier()
        pltpu.sync_copy(dacc.at[pl.ds(s * rpt, rpt)],
                        degp.at[c, pl.ds(s * rpt, rpt)])

        iota16 = lax.iota(jnp.int32, 16)

        def rep_chunk(drep):
            def one(q, _):
                nb = s * rpt + q * RC
                pltpu.sync_copy(dacc.at[pl.ds(nb, RC)], dbuf)

                def grp(k, _):
                    v = dbuf[pl.ds(k * 16, 16)]
                    for a in range(16):
                        idx = iota16 * F + (k * 16 * F + a)
                        plsc.store_scatter(repbuf, [idx], v)
                    return 0

                lax.fori_loop(0, RC // 16, grp, 0)
                pltpu.sync_copy(repbuf, drep.at[pl.ds(nb * F, RC * F)])
                return 0

            lax.fori_loop(0, rpt // RC, one, 0)

        @pl.when(c == 0)
        def _():
            rep_chunk(d0rep)

        @pl.when(c == 1)
        def _():
            rep_chunk(d1rep)

    return setup


def _sc_layer(E, n, N_PAD, C=800):
    ept = E // NS
    niter = ept // C
    npair = (niter - 1) // 2
    assert niter == 2 * npair + 1
    n_acc = ((n + 1 + 127) // 128) * 128
    rpt = n_acc // NS
    mesh = plsc.VectorSubcoreMesh(core_axis_name="c", subcore_axis_name="s")

    @functools.partial(
        pl.kernel,
        out_type=(
            jax.ShapeDtypeStruct((N_PAD, F), jnp.float32),
            jax.ShapeDtypeStruct((N_PAD, F), jnp.float32),
        ),
        mesh=mesh,
        scratch_types=[
            pltpu.VMEM((C,), jnp.int32),
            pltpu.VMEM((C,), jnp.int32),
            pltpu.VMEM((C,), jnp.int32),
            pltpu.VMEM((C,), jnp.int32),
            pltpu.VMEM((C, F), jnp.float32),
            pltpu.VMEM((C, F), jnp.float32),
            pltpu.VMEM_SHARED((n_acc, F), jnp.float32),
            pltpu.SemaphoreType.DMA,
            pltpu.SemaphoreType.DMA,
            pltpu.SemaphoreType.DMA,
            pltpu.SemaphoreType.DMA,
        ],
        compiler_params=pltpu.CompilerParams(use_tc_tiling_on_sc=False),
    )
    def layer(row, colp, ulo, uhi, alo, ahi,
              rbA, cbA, rbB, cbB, gbA, gbB, acc, siA, siB, sgA, sgB):
        c = lax.axis_index("c")
        s = lax.axis_index("s")

        def zrow(i, _):
            gbA[i, :] = jnp.zeros((F,), jnp.float32)
            return 0

        lax.fori_loop(0, C, zrow, 0)
        base = s * rpt
        done = 0
        while done < rpt:
            step = min(C, rpt - done)
            pltpu.sync_copy(gbA.at[pl.ds(0, step)],
                            acc.at[pl.ds(base + done, step)])
            done += step
        plsc.subcore_barrier()

        def run(u_hbm):
            tb = s * ept

            def issue_idx(k, rb, cb, sem):
                pltpu.async_copy(row.at[pl.ds(tb + k * C, C)], rb, sem)
                pltpu.async_copy(colp.at[pl.ds(tb + k * C, C)], cb, sem)

            def wait_idx(rb, cb, sem):
                pltpu.make_async_copy(row.at[pl.ds(tb, C)], rb, sem).wait()
                pltpu.make_async_copy(colp.at[pl.ds(tb, C)], cb, sem).wait()

            def wait_gather(rb, gb, sem):
                pltpu.make_async_copy(u_hbm.at[rb], gb, sem).wait()

            issue_idx(0, rbA, cbA, siA)
            issue_idx(1, rbB, cbB, siB)
            wait_idx(rbA, cbA, siA)
            pltpu.async_copy(u_hbm.at[rbA], gbA, sgA)

            def pair(j, _):
                wait_idx(rbB, cbB, siB)
                wait_gather(rbA, gbA, sgA)
                pltpu.async_copy(u_hbm.at[rbB], gbB, sgB)
                pltpu.sync_copy(gbA, acc.at[cbA], add=True)
                issue_idx(2 * j + 2, rbA, cbA, siA)
                wait_gather(rbB, gbB, sgB)
                wait_idx(rbA, cbA, siA)
                pltpu.async_copy(u_hbm.at[rbA], gbA, sgA)
                pltpu.sync_copy(gbB, acc.at[cbB], add=True)

                @pl.when(j < npair - 1)
                def _():
                    issue_idx(2 * j + 3, rbB, cbB, siB)

                return 0

            lax.fori_loop(0, npair, pair, 0)
            wait_gather(rbA, gbA, sgA)
            pltpu.sync_copy(gbA, acc.at[cbA], add=True)

        @pl.when(c == 0)
        def _():
            run(ulo)

        @pl.when(c == 1)
        def _():
            run(uhi)

        plsc.subcore_barrier()

        @pl.when(c == 0)
        def _():
            pltpu.sync_copy(acc.at[pl.ds(s * rpt, rpt)],
                            alo.at[pl.ds(s * rpt, rpt)])

        @pl.when(c == 1)
        def _():
            pltpu.sync_copy(acc.at[pl.ds(s * rpt, rpt)],
                            ahi.at[pl.ds(s * rpt, rpt)])

    return layer


def _dis_of(degp_blk):
    deg = degp_blk[0, :] + degp_blk[1, :] + 2.0
    return lax.rsqrt(deg)[:, None]


def _row_mask(n):
    rows = pl.program_id(0) * BN + lax.broadcasted_iota(jnp.int32, (BN, 1), 0)
    return rows < n


def _enc1_body(n, x_ref, w_ref, b_ref, u_ref, st_ref):
    u = jnp.dot(x_ref[...], w_ref[...], precision=_HIGH,
                preferred_element_type=jnp.float32) + b_ref[...]
    u_ref[...] = u
    um = jnp.where(_row_mask(n), u, 0.0)
    st = jnp.stack([jnp.sum(um, axis=0), jnp.sum(um * um, axis=0)])

    @pl.when(pl.program_id(0) == 0)
    def _():
        st_ref[...] = st

    @pl.when(pl.program_id(0) > 0)
    def _():
        st_ref[...] += st


def _bn_elu(u, st, g, be, n):
    mean = st[0:1, :] / n
    var = st[1:2, :] / n - mean * mean
    h = (u - mean) * lax.rsqrt(var + 0.001) * g + be
    return jnp.where(h > 0, h, jnp.exp(h) - 1.0)


def _enc2_body(n, u_ref, st_ref, g_ref, be_ref, w_ref, b_ref, d0_ref, d1_ref,
               v_ref, st2_ref, dis_ref):
    h = _bn_elu(u_ref[...], st_ref[...], g_ref[...], be_ref[...], n)
    v = jnp.dot(h, w_ref[...], precision=_HIGH,
                preferred_element_type=jnp.float32) + b_ref[...]
    v_ref[...] = v
    dis_ref[...] = lax.rsqrt(d0_ref[...] + d1_ref[...] + 2.0)
    vm = jnp.where(_row_mask(n), v, 0.0)
    st = jnp.stack([jnp.sum(vm, axis=0), jnp.sum(vm * vm, axis=0)])

    @pl.when(pl.program_id(0) == 0)
    def _():
        st2_ref[...] = st

    @pl.when(pl.program_id(0) > 0)
    def _():
        st2_ref[...] += st


def _mm1_body(n, v_ref, st_ref, g_ref, be_ref, degp_ref, w_ref,
              ulo_ref, uhi_ref):
    h = _bn_elu(v_ref[...], st_ref[...], g_ref[...], be_ref[...], n)
    t = jnp.dot(h, w_ref[...], precision=_HIGH,
                preferred_element_type=jnp.float32)
    u = _dis_of(degp_ref[...]) * t
    ulo_ref[...] = u[:, :F]
    uhi_ref[...] = u[:, F:]


def _mid_body(relu, alo_ref, ahi_ref, ulo_ref, uhi_ref, dis_ref,
              bl_ref, bh_ref, kll_ref, khl_ref, klh_ref, khh_ref,
              olo_ref, ohi_ref):
    dis = dis_ref[...]
    zl = dis * alo_ref[...] + 2.0 * dis * ulo_ref[...] + bl_ref[...]
    zh = dis * ahi_ref[...] + 2.0 * dis * uhi_ref[...] + bh_ref[...]
    if relu:
        zl = jnp.maximum(zl, 0.0)
        zh = jnp.maximum(zh, 0.0)
    dot = functools.partial(jnp.dot, precision=_HIGH,
                            preferred_element_type=jnp.float32)
    olo_ref[...] = dis * (dot(zl, kll_ref[...]) + dot(zh, khl_ref[...]))
    ohi_ref[...] = dis * (dot(zl, klh_ref[...]) + dot(zh, khh_ref[...]))


def _fin_body(alo_ref, ahi_ref, ulo_ref, uhi_ref, dis_ref, bl_ref, bh_ref,
              olo_ref, ohi_ref):
    dis = dis_ref[...]
    olo_ref[...] = dis * alo_ref[...] + 2.0 * dis * ulo_ref[...] + bl_ref[...]
    ohi_ref[...] = dis * ahi_ref[...] + 2.0 * dis * uhi_ref[...] + bh_ref[...]


def _row_spec(cols):
    return pl.BlockSpec((BN, cols), lambda i: (i, 0))


def _full_spec(shape):
    return pl.BlockSpec(shape, lambda i: tuple(0 for _ in shape))


def _degp_spec():
    return pl.BlockSpec((NC, BN), lambda i: (0, i))


def kernel(x, edge_index, W1, b1, g1, be1, W2, b2, g2, be2,
           Wg1, bg1, Wg2, bg2, Wg3, bg3):
    n, d = x.shape
    e = edge_index.shape[1]
    n_pad = _npad(n)
    d1 = W1.shape[1]
    d2 = W2.shape[1]
    grid = (n_pad // BN,)

    m8 = n_pad // 8
    mb = BN // 8
    b1r, g1r, be1r = b1[None, :], g1[None, :], be1[None, :]
    b2r, g2r, be2r = b2[None, :], g2[None, :], be2[None, :]
    i8 = jnp.eye(8, dtype=jnp.float32)

    def krons(w):
        return (jnp.kron(i8, w[:F, :F]), jnp.kron(i8, w[F:, :F]),
                jnp.kron(i8, w[:F, F:]), jnp.kron(i8, w[F:, F:]))

    def btiles(b):
        return jnp.tile(b[:F], 8)[None, :], jnp.tile(b[F:], 8)[None, :]

    xp = jnp.pad(x, ((0, n_pad - n), (0, 0)))
    row = edge_index[0]
    col = edge_index[1]

    degp, colp, d0rep, d1rep = _sc_setup(e, n_pad, n)(row, col)
    d0p = jnp.reshape(d0rep, (m8, 128))
    d1p = jnp.reshape(d1rep, (m8, 128))

    U, st1 = pl.pallas_call(
        functools.partial(_enc1_body, float(n)),
        grid=grid,
        in_specs=[_row_spec(d), _full_spec((d, d1)), _full_spec((1, d1))],
        out_specs=[_row_spec(d1), _full_spec((2, d1))],
        out_shape=[jax.ShapeDtypeStruct((n_pad, d1), jnp.float32),
                   jax.ShapeDtypeStruct((2, d1), jnp.float32)],
        compiler_params=pltpu.CompilerParams(
            dimension_semantics=("arbitrary",)),
    )(xp, W1, b1r)

    V, st2, dis_p = pl.pallas_call(
        functools.partial(_enc2_body, float(n)),
        grid=grid,
        in_specs=[_row_spec(d1), _full_spec((2, d1)),
                  _full_spec((1, d1)), _full_spec((1, d1)),
                  _full_spec((d1, d2)), _full_spec((1, d2)),
                  pl.BlockSpec((mb, 128), lambda i: (i, 0)),
                  pl.BlockSpec((mb, 128), lambda i: (i, 0))],
        out_specs=[_row_spec(d2), _full_spec((2, d2)),
                   pl.BlockSpec((mb, 128), lambda i: (i, 0))],
        out_shape=[jax.ShapeDtypeStruct((n_pad, d2), jnp.float32),
                   jax.ShapeDtypeStruct((2, d2), jnp.float32),
                   jax.ShapeDtypeStruct((m8, 128), jnp.float32)],
        compiler_params=pltpu.CompilerParams(
            dimension_semantics=("arbitrary",)),
    )(U, st1, g1r, be1r, W2, b2r, d0p, d1p)

    ulo, uhi = pl.pallas_call(
        functools.partial(_mm1_body, float(n)),
        grid=grid,
        in_specs=[_row_spec(d2), _full_spec((2, d2)),
                  _full_spec((1, d2)), _full_spec((1, d2)),
                  _degp_spec(), _full_spec((d2, d2))],
        out_specs=[_row_spec(F), _row_spec(F)],
        out_shape=[jax.ShapeDtypeStruct((n_pad, F), jnp.float32),
                   jax.ShapeDtypeStruct((n_pad, F), jnp.float32)],
        compiler_params=pltpu.CompilerParams(
            dimension_semantics=("parallel",)),
    )(V, st2, g2r, be2r, degp, Wg1)

    sc_layer = _sc_layer(e, n, n_pad)
    pspec = pl.BlockSpec((mb, 128), lambda i: (i, 0))

    def mid(relu, alo_p, ahi_p, ulo_p, uhi_p, bprev, wg):
        kll, khl, klh, khh = krons(wg)
        bl, bh = btiles(bprev)
        return pl.pallas_call(
            functools.partial(_mid_body, relu),
            grid=grid,
            in_specs=[pspec, pspec, pspec, pspec, pspec,
                      _full_spec((1, 128)), _full_spec((1, 128)),
                      _full_spec((128, 128)), _full_spec((128, 128)),
                      _full_spec((128, 128)), _full_spec((128, 128))],
            out_specs=[pspec, pspec],
            out_shape=[jax.ShapeDtypeStruct((m8, 128), jnp.float32),
                       jax.ShapeDtypeStruct((m8, 128), jnp.float32)],
            compiler_params=pltpu.CompilerParams(
                dimension_semantics=("parallel",)),
        )(alo_p, ahi_p, ulo_p, uhi_p, dis_p, bl, bh, kll, khl, klh, khh)

    def as_pack(a_lin):
        return jnp.reshape(a_lin, (m8, 128))

    def as_lin(a_p):
        return jnp.reshape(a_p, (n_pad, F))

    u1lo_p = lax.optimization_barrier(as_pack(ulo))
    u1hi_p = lax.optimization_barrier(as_pack(uhi))

    alo1, ahi1 = sc_layer(row, colp, as_lin(u1lo_p), as_lin(u1hi_p))
    ulo2_p, uhi2_p = mid(True, as_pack(alo1), as_pack(ahi1),
                         u1lo_p, u1hi_p, bg1, Wg2)
    alo2, ahi2 = sc_layer(row, colp, as_lin(ulo2_p), as_lin(uhi2_p))
    ulo3_p, uhi3_p = mid(False, as_pack(alo2), as_pack(ahi2),
                         ulo2_p, uhi2_p, bg2, Wg3)
    alo3, ahi3 = sc_layer(row, colp, as_lin(ulo3_p), as_lin(uhi3_p))

    bl3, bh3 = btiles(bg3)
    zlo_p, zhi_p = pl.pallas_call(
        _fin_body,
        grid=grid,
        in_specs=[pspec, pspec, pspec, pspec, pspec,
                  _full_spec((1, 128)), _full_spec((1, 128))],
        out_specs=[pspec, pspec],
        out_shape=[jax.ShapeDtypeStruct((m8, 128), jnp.float32),
                   jax.ShapeDtypeStruct((m8, 128), jnp.float32)],
        compiler_params=pltpu.CompilerParams(
            dimension_semantics=("parallel",)),
    )(as_pack(alo3), as_pack(ahi3), ulo3_p, uhi3_p, dis_p, bl3, bh3)

    h3 = jnp.concatenate([jnp.reshape(zlo_p, (n_pad, F)),
                          jnp.reshape(zhi_p, (n_pad, F))], axis=1)
    return h3[:n]


# speedup vs baseline: 40.8119x; 1.0001x over previous
import functools

import jax
import jax.numpy as jnp
from jax import lax
from jax.experimental import pallas as pl
from jax.experimental.pallas import tpu as pltpu
from jax.experimental.pallas import tpu_sc as plsc

NC = 2
NS = 16
F = 16
BN = 4096

_HIGH = lax.Precision.HIGHEST


def _npad(n):
    return ((n + 1 + BN - 1) // BN) * BN


def _sc_setup(E, N_PAD, trash, C=2000):
    epw = E // (NC * NS)
    niter = epw // C
    rpt = N_PAD // NS
    mesh = plsc.VectorSubcoreMesh(core_axis_name="c", subcore_axis_name="s")

    RC = 800

    @functools.partial(
        pl.kernel,
        out_type=(
            jax.ShapeDtypeStruct((NC, N_PAD), jnp.float32),
            jax.ShapeDtypeStruct((E,), jnp.int32),
            jax.ShapeDtypeStruct((N_PAD * F,), jnp.float32),
            jax.ShapeDtypeStruct((N_PAD * F,), jnp.float32),
        ),
        mesh=mesh,
        scratch_types=[
            pltpu.VMEM((C,), jnp.int32),
            pltpu.VMEM((C,), jnp.int32),
            pltpu.VMEM((C,), jnp.float32),
            pltpu.VMEM((C,), jnp.int32),
            pltpu.VMEM((rpt,), jnp.float32),
            pltpu.VMEM((RC,), jnp.float32),
            pltpu.VMEM((RC * F,), jnp.float32),
            pltpu.VMEM_SHARED((N_PAD,), jnp.float32),
        ],
        compiler_params=pltpu.CompilerParams(needs_layout_passes=False),
    )
    def setup(row, col, degp, colp, d0rep, d1rep,
              rbuf, cbuf, wbuf, cpbuf, zbuf, dbuf, repbuf, dacc):
        c = lax.axis_index("c")
        s = lax.axis_index("s")
        w = s * NC + c

        def zrow(i, _):
            zbuf[pl.ds(i * 16, 16)] = jnp.zeros((16,), jnp.float32)
            return 0

        lax.fori_loop(0, rpt // 16, zrow, 0)
        pltpu.sync_copy(zbuf, dacc.at[pl.ds(s * rpt, rpt)])
        plsc.subcore_barrier()

        def body(i, _):
            base = w * epw + i * C
            pltpu.sync_copy(row.at[pl.ds(base, C)], rbuf)
            pltpu.sync_copy(col.at[pl.ds(base, C)], cbuf)

            def vec(k, _):
                sl = pl.ds(k * 16, 16)
                r = rbuf[sl]
                cc = cbuf[sl]
                m = r == cc
                wbuf[sl] = jnp.where(m, 0.0, 1.0).astype(jnp.float32)
                cpbuf[sl] = jnp.where(m, trash, cc)
                return 0

            lax.fori_loop(0, C // 16, vec, 0)
            pltpu.sync_copy(wbuf, dacc.at[rbuf], add=True)
            pltpu.sync_copy(cpbuf, colp.at[pl.ds(base, C)])
            return 0

        lax.fori_loop(0, niter, body, 0)
        plsc.subcore_barrier()
        pltpu.sync_copy(dacc.at[pl.ds(s * rpt, rpt)],
                        degp.at[c, pl.ds(s * rpt, rpt)])

        iota16 = lax.iota(jnp.int32, 16)

        def rep_chunk(drep):
            def one(q, _):
                nb = s * rpt + q * RC
                pltpu.sync_copy(dacc.at[pl.ds(nb, RC)], dbuf)

                def grp(k, _):
                    v = dbuf[pl.ds(k * 16, 16)]
                    for a in range(16):
                        idx = iota16 * F + (k * 16 * F + a)
                        plsc.store_scatter(repbuf, [idx], v)
                    return 0

                lax.fori_loop(0, RC // 16, grp, 0)
                pltpu.sync_copy(repbuf, drep.at[pl.ds(nb * F, RC * F)])
                return 0

            lax.fori_loop(0, rpt // RC, one, 0)

        @pl.when(c == 0)
        def _():
            rep_chunk(d0rep)

        @pl.when(c == 1)
        def _():
            rep_chunk(d1rep)

    return setup


def _sc_layer(E, n, N_PAD, C=800):
    ept = E // NS
    niter = ept // C
    nblk = niter // 6
    head = min(6, niter)
    n_acc = ((n + 1 + 127) // 128) * 128
    rpt = n_acc // NS
    mesh = plsc.VectorSubcoreMesh(core_axis_name="c", subcore_axis_name="s")

    @functools.partial(
        pl.kernel,
        out_type=(
            jax.ShapeDtypeStruct((N_PAD, F), jnp.float32),
            jax.ShapeDtypeStruct((N_PAD, F), jnp.float32),
        ),
        mesh=mesh,
        scratch_types=[
            pltpu.VMEM((C,), jnp.int32),
            pltpu.VMEM((C,), jnp.int32),
            pltpu.VMEM((C,), jnp.int32),
            pltpu.VMEM((C,), jnp.int32),
            pltpu.VMEM((C,), jnp.int32),
            pltpu.VMEM((C,), jnp.int32),
            pltpu.VMEM((C, F), jnp.float32),
            pltpu.VMEM((C, F), jnp.float32),
            pltpu.VMEM_SHARED((n_acc, F), jnp.float32),
            pltpu.SemaphoreType.DMA,
            pltpu.SemaphoreType.DMA,
            pltpu.SemaphoreType.DMA,
            pltpu.SemaphoreType.DMA,
            pltpu.SemaphoreType.DMA,
            pltpu.SemaphoreType.DMA,
            pltpu.SemaphoreType.DMA,
        ],
        compiler_params=pltpu.CompilerParams(use_tc_tiling_on_sc=False),
    )
    def layer(row, colp, ulo, uhi, alo, ahi,
              rb0, cb0, rb1, cb1, rb2, cb2, gb0, gb1, acc,
              si0, si1, si2, sg0, sg1, ss0, ss1):
        c = lax.axis_index("c")
        s = lax.axis_index("s")
        rb, cb, si = (rb0, rb1, rb2), (cb0, cb1, cb2), (si0, si1, si2)
        gb, sg, ss = (gb0, gb1), (sg0, sg1), (ss0, ss1)

        def zrow(i, _):
            gb0[i, :] = jnp.zeros((F,), jnp.float32)
            return 0

        lax.fori_loop(0, C, zrow, 0)
        base = s * rpt
        done = 0
        while done < rpt:
            step = min(C, rpt - done)
            pltpu.sync_copy(gb0.at[pl.ds(0, step)],
                            acc.at[pl.ds(base + done, step)])
            done += step
        plsc.subcore_barrier()

        def run(u_hbm):
            tb = s * ept

            def issue_idx(k, j):
                pltpu.async_copy(row.at[pl.ds(tb + k * C, C)], rb[j], si[j])
                pltpu.async_copy(colp.at[pl.ds(tb + k * C, C)], cb[j], si[j])

            def wait_idx(j):
                pltpu.make_async_copy(row.at[pl.ds(tb, C)], rb[j],
                                      si[j]).wait()
                pltpu.make_async_copy(colp.at[pl.ds(tb, C)], cb[j],
                                      si[j]).wait()

            def issue_gather(j, g):
                pltpu.async_copy(u_hbm.at[rb[j]], gb[g], sg[g])

            def wait_gather(j, g):
                pltpu.make_async_copy(u_hbm.at[rb[j]], gb[g], sg[g]).wait()

            def issue_scat(j, g):
                pltpu.async_copy(gb[g], acc.at[cb[j]], ss[g], add=True)

            def wait_scat(j, g):
                pltpu.make_async_copy(gb[g], acc.at[cb[j]], ss[g]).wait()

            def steps(k0, ks, static):
                for t in range(ks):
                    j, jn = t % 3, (t + 1) % 3
                    g, gn = t % 2, (t + 1) % 2
                    k = k0 + t
                    wait_gather(j, g)
                    issue_scat(j, g)
                    if (not static) or k + 1 < niter:
                        wait_idx(jn)
                    if (not static) or k > 0:
                        wait_scat((t + 2) % 3, gn)
                    if (not static) or k + 1 < niter:
                        issue_gather(jn, gn)
                    if (not static) or k + 2 < niter:
                        issue_idx(k + 2, (t + 2) % 3)

            issue_idx(0, 0)
            issue_idx(1, 1)
            wait_idx(0)
            issue_gather(0, 0)
            steps(0, head, True)

            if nblk > 1:
                def body(b, _):
                    steps(6 * b, 6, False)
                    return 0

                lax.fori_loop(1, nblk, body, 0)
            for k in range(6 * nblk, niter):
                t = k % 6
                j, jn = t % 3, (t + 1) % 3
                g, gn = t % 2, (t + 1) % 2
                wait_gather(j, g)
                issue_scat(j, g)
                if k + 1 < niter:
                    wait_idx(jn)
                wait_scat((t + 2) % 3, gn)
                if k + 1 < niter:
                    issue_gather(jn, gn)
                if k + 2 < niter:
                    issue_idx(k + 2, (t + 2) % 3)
            lt = (niter - 1) % 6
            wait_scat(lt % 3, lt % 2)

        @pl.when(c == 0)
        def _():
            run(ulo)

        @pl.when(c == 1)
        def _():
            run(uhi)

        plsc.subcore_barrier()

        @pl.when(c == 0)
        def _():
            pltpu.sync_copy(acc.at[pl.ds(s * rpt, rpt)],
                            alo.at[pl.ds(s * rpt, rpt)])

        @pl.when(c == 1)
        def _():
            pltpu.sync_copy(acc.at[pl.ds(s * rpt, rpt)],
                            ahi.at[pl.ds(s * rpt, rpt)])

    return layer


def _dis_of(degp_blk):
    deg = degp_blk[0, :] + degp_blk[1, :] + 2.0
    return lax.rsqrt(deg)[:, None]


def _row_mask(n):
    rows = pl.program_id(0) * BN + lax.broadcasted_iota(jnp.int32, (BN, 1), 0)
    return rows < n


def _enc1_body(n, x_ref, w_ref, b_ref, u_ref, st_ref):
    u = jnp.dot(x_ref[...], w_ref[...], precision=_HIGH,
                preferred_element_type=jnp.float32) + b_ref[...]
    u_ref[...] = u
    um = jnp.where(_row_mask(n), u, 0.0)
    st = jnp.stack([jnp.sum(um, axis=0), jnp.sum(um * um, axis=0)])

    @pl.when(pl.program_id(0) == 0)
    def _():
        st_ref[...] = st

    @pl.when(pl.program_id(0) > 0)
    def _():
        st_ref[...] += st


def _bn_elu(u, st, g, be, n):
    mean = st[0:1, :] / n
    var = st[1:2, :] / n - mean * mean
    h = (u - mean) * lax.rsqrt(var + 0.001) * g + be
    return jnp.where(h > 0, h, jnp.exp(h) - 1.0)


def _enc2_body(n, u_ref, st_ref, g_ref, be_ref, w_ref, b_ref, d0_ref, d1_ref,
               v_ref, st2_ref, dis_ref):
    h = _bn_elu(u_ref[...], st_ref[...], g_ref[...], be_ref[...], n)
    v = jnp.dot(h, w_ref[...], precision=_HIGH,
                preferred_element_type=jnp.float32) + b_ref[...]
    v_ref[...] = v
    dis_ref[...] = lax.rsqrt(d0_ref[...] + d1_ref[...] + 2.0)
    vm = jnp.where(_row_mask(n), v, 0.0)
    st = jnp.stack([jnp.sum(vm, axis=0), jnp.sum(vm * vm, axis=0)])

    @pl.when(pl.program_id(0) == 0)
    def _():
        st2_ref[...] = st

    @pl.when(pl.program_id(0) > 0)
    def _():
        st2_ref[...] += st


def _mm1_body(n, v_ref, st_ref, g_ref, be_ref, degp_ref, w_ref,
              ulo_ref, uhi_ref):
    h = _bn_elu(v_ref[...], st_ref[...], g_ref[...], be_ref[...], n)
    t = jnp.dot(h, w_ref[...], precision=_HIGH,
                preferred_element_type=jnp.float32)
    u = _dis_of(degp_ref[...]) * t
    ulo_ref[...] = u[:, :F]
    uhi_ref[...] = u[:, F:]


def _mid_body(relu, alo_ref, ahi_ref, ulo_ref, uhi_ref, dis_ref,
              bl_ref, bh_ref, kll_ref, khl_ref, klh_ref, khh_ref,
              olo_ref, ohi_ref):
    dis = dis_ref[...]
    zl = dis * alo_ref[...] + 2.0 * dis * ulo_ref[...] + bl_ref[...]
    zh = dis * ahi_ref[...] + 2.0 * dis * uhi_ref[...] + bh_ref[...]
    if relu:
        zl = jnp.maximum(zl, 0.0)
        zh = jnp.maximum(zh, 0.0)
    dot = functools.partial(jnp.dot, precision=_HIGH,
                            preferred_element_type=jnp.float32)
    olo_ref[...] = dis * (dot(zl, kll_ref[...]) + dot(zh, khl_ref[...]))
    ohi_ref[...] = dis * (dot(zl, klh_ref[...]) + dot(zh, khh_ref[...]))


def _fin_body(alo_ref, ahi_ref, ulo_ref, uhi_ref, dis_ref, bl_ref, bh_ref,
              olo_ref, ohi_ref):
    dis = dis_ref[...]
    olo_ref[...] = dis * alo_ref[...] + 2.0 * dis * ulo_ref[...] + bl_ref[...]
    ohi_ref[...] = dis * ahi_ref[...] + 2.0 * dis * uhi_ref[...] + bh_ref[...]


def _row_spec(cols):
    return pl.BlockSpec((BN, cols), lambda i: (i, 0))


def _full_spec(shape):
    return pl.BlockSpec(shape, lambda i: tuple(0 for _ in shape))


def _degp_spec():
    return pl.BlockSpec((NC, BN), lambda i: (0, i))


def kernel(x, edge_index, W1, b1, g1, be1, W2, b2, g2, be2,
           Wg1, bg1, Wg2, bg2, Wg3, bg3):
    n, d = x.shape
    e = edge_index.shape[1]
    n_pad = _npad(n)
    d1 = W1.shape[1]
    d2 = W2.shape[1]
    grid = (n_pad // BN,)

    m8 = n_pad // 8
    mb = BN // 8
    b1r, g1r, be1r = b1[None, :], g1[None, :], be1[None, :]
    b2r, g2r, be2r = b2[None, :], g2[None, :], be2[None, :]
    i8 = jnp.eye(8, dtype=jnp.float32)

    def krons(w):
        return (jnp.kron(i8, w[:F, :F]), jnp.kron(i8, w[F:, :F]),
                jnp.kron(i8, w[:F, F:]), jnp.kron(i8, w[F:, F:]))

    def btiles(b):
        return jnp.tile(b[:F], 8)[None, :], jnp.tile(b[F:], 8)[None, :]

    xp = jnp.pad(x, ((0, n_pad - n), (0, 0)))
    row = edge_index[0]
    col = edge_index[1]

    degp, colp, d0rep, d1rep = _sc_setup(e, n_pad, n)(row, col)
    d0p = jnp.reshape(d0rep, (m8, 128))
    d1p = jnp.reshape(d1rep, (m8, 128))

    U, st1 = pl.pallas_call(
        functools.partial(_enc1_body, float(n)),
        grid=grid,
        in_specs=[_row_spec(d), _full_spec((d, d1)), _full_spec((1, d1))],
        out_specs=[_row_spec(d1), _full_spec((2, d1))],
        out_shape=[jax.ShapeDtypeStruct((n_pad, d1), jnp.float32),
                   jax.ShapeDtypeStruct((2, d1), jnp.float32)],
        compiler_params=pltpu.CompilerParams(
            dimension_semantics=("arbitrary",)),
    )(xp, W1, b1r)

    V, st2, dis_p = pl.pallas_call(
        functools.partial(_enc2_body, float(n)),
        grid=grid,
        in_specs=[_row_spec(d1), _full_spec((2, d1)),
                  _full_spec((1, d1)), _full_spec((1, d1)),
                  _full_spec((d1, d2)), _full_spec((1, d2)),
                  pl.BlockSpec((mb, 128), lambda i: (i, 0)),
                  pl.BlockSpec((mb, 128), lambda i: (i, 0))],
        out_specs=[_row_spec(d2), _full_spec((2, d2)),
                   pl.BlockSpec((mb, 128), lambda i: (i, 0))],
        out_shape=[jax.ShapeDtypeStruct((n_pad, d2), jnp.float32),
                   jax.ShapeDtypeStruct((2, d2), jnp.float32),
                   jax.ShapeDtypeStruct((m8, 128), jnp.float32)],
        compiler_params=pltpu.CompilerParams(
            dimension_semantics=("arbitrary",)),
    )(U, st1, g1r, be1r, W2, b2r, d0p, d1p)

    ulo, uhi = pl.pallas_call(
        functools.partial(_mm1_body, float(n)),
        grid=grid,
        in_specs=[_row_spec(d2), _full_spec((2, d2)),
                  _full_spec((1, d2)), _full_spec((1, d2)),
                  _degp_spec(), _full_spec((d2, d2))],
        out_specs=[_row_spec(F), _row_spec(F)],
        out_shape=[jax.ShapeDtypeStruct((n_pad, F), jnp.float32),
                   jax.ShapeDtypeStruct((n_pad, F), jnp.float32)],
        compiler_params=pltpu.CompilerParams(
            dimension_semantics=("parallel",)),
    )(V, st2, g2r, be2r, degp, Wg1)

    sc_layer = _sc_layer(e, n, n_pad)
    pspec = pl.BlockSpec((mb, 128), lambda i: (i, 0))

    def mid(relu, alo_p, ahi_p, ulo_p, uhi_p, bprev, wg):
        kll, khl, klh, khh = krons(wg)
        bl, bh = btiles(bprev)
        return pl.pallas_call(
            functools.partial(_mid_body, relu),
            grid=grid,
            in_specs=[pspec, pspec, pspec, pspec, pspec,
                      _full_spec((1, 128)), _full_spec((1, 128)),
                      _full_spec((128, 128)), _full_spec((128, 128)),
                      _full_spec((128, 128)), _full_spec((128, 128))],
            out_specs=[pspec, pspec],
            out_shape=[jax.ShapeDtypeStruct((m8, 128), jnp.float32),
                       jax.ShapeDtypeStruct((m8, 128), jnp.float32)],
            compiler_params=pltpu.CompilerParams(
                dimension_semantics=("parallel",)),
        )(alo_p, ahi_p, ulo_p, uhi_p, dis_p, bl, bh, kll, khl, klh, khh)

    def as_pack(a_lin):
        return jnp.reshape(a_lin, (m8, 128))

    def as_lin(a_p):
        return jnp.reshape(a_p, (n_pad, F))

    u1lo_p = lax.optimization_barrier(as_pack(ulo))
    u1hi_p = lax.optimization_barrier(as_pack(uhi))

    alo1, ahi1 = sc_layer(row, colp, as_lin(u1lo_p), as_lin(u1hi_p))
    ulo2_p, uhi2_p = mid(True, as_pack(alo1), as_pack(ahi1),
                         u1lo_p, u1hi_p, bg1, Wg2)
    alo2, ahi2 = sc_layer(row, colp, as_lin(ulo2_p), as_lin(uhi2_p))
    ulo3_p, uhi3_p = mid(False, as_pack(alo2), as_pack(ahi2),
                         ulo2_p, uhi2_p, bg2, Wg3)
    alo3, ahi3 = sc_layer(row, colp, as_lin(ulo3_p), as_lin(uhi3_p))

    bl3, bh3 = btiles(bg3)
    zlo_p, zhi_p = pl.pallas_call(
        _fin_body,
        grid=grid,
        in_specs=[pspec, pspec, pspec, pspec, pspec,
                  _full_spec((1, 128)), _full_spec((1, 128))],
        out_specs=[pspec, pspec],
        out_shape=[jax.ShapeDtypeStruct((m8, 128), jnp.float32),
                   jax.ShapeDtypeStruct((m8, 128), jnp.float32)],
        compiler_params=pltpu.CompilerParams(
            dimension_semantics=("parallel",)),
    )(as_pack(alo3), as_pack(ahi3), ulo3_p, uhi3_p, dis_p, bl3, bh3)

    h3 = jnp.concatenate([jnp.reshape(zlo_p, (n_pad, F)),
                          jnp.reshape(zhi_p, (n_pad, F))], axis=1)
    return h3[:n]


# speedup vs baseline: 45.0142x; 1.1030x over previous
import functools

import jax
import jax.numpy as jnp
from jax import lax
from jax.experimental import pallas as pl
from jax.experimental.pallas import tpu as pltpu
from jax.experimental.pallas import tpu_sc as plsc

NC = 2
NS = 16
F = 16
BN = 4096

_HIGH = lax.Precision.DEFAULT


def _npad(n):
    return ((n + 1 + BN - 1) // BN) * BN


def _sc_setup(E, N_PAD, trash, C=2560, NWS=25):
    epw = E // NWS
    niter = epw // C
    rpt = N_PAD // NS
    mesh = plsc.VectorSubcoreMesh(core_axis_name="c", subcore_axis_name="s")

    RC = 800

    @functools.partial(
        pl.kernel,
        out_type=(
            jax.ShapeDtypeStruct((NC, N_PAD), jnp.float32),
            jax.ShapeDtypeStruct((E,), jnp.int32),
            jax.ShapeDtypeStruct((E,), jnp.int32),
            jax.ShapeDtypeStruct((N_PAD * F,), jnp.float32),
            jax.ShapeDtypeStruct((N_PAD * F,), jnp.float32),
        ),
        mesh=mesh,
        scratch_types=[
            pltpu.VMEM((2, C), jnp.int32),
            pltpu.VMEM((C,), jnp.int32),
            pltpu.VMEM((C,), jnp.float32),
            pltpu.VMEM((C,), jnp.int32),
            pltpu.VMEM((rpt,), jnp.float32),
            pltpu.VMEM((RC,), jnp.float32),
            pltpu.VMEM((RC * F,), jnp.float32),
            pltpu.VMEM_SHARED((N_PAD,), jnp.float32),
        ],
        compiler_params=pltpu.CompilerParams(needs_layout_passes=False),
    )
    def setup(ei, degp, colp, rowc, d0rep, d1rep,
              ebuf, rbuf, wbuf, cpbuf, zbuf, dbuf, repbuf, dacc):
        c = lax.axis_index("c")
        s = lax.axis_index("s")
        w = s * NC + c

        def zrow(i, _):
            zbuf[pl.ds(i * 16, 16)] = jnp.zeros((16,), jnp.float32)
            return 0

        lax.fori_loop(0, rpt // 16, zrow, 0)
        pltpu.sync_copy(zbuf, dacc.at[pl.ds(s * rpt, rpt)])
        plsc.subcore_barrier()

        def body(i, _):
            base = w * epw + i * C
            pltpu.sync_copy(ei.at[:, pl.ds(base, C)], ebuf)

            def vec(k, _):
                sl = pl.ds(k * 16, 16)
                r = ebuf[0, sl]
                cc = ebuf[1, sl]
                m = r == cc
                rbuf[sl] = r
                wbuf[sl] = jnp.where(m, 0.0, 1.0).astype(jnp.float32)
                cpbuf[sl] = jnp.where(m, trash, cc)
                return 0

            lax.fori_loop(0, C // 16, vec, 0)
            pltpu.sync_copy(wbuf, dacc.at[rbuf], add=True)
            pltpu.sync_copy(cpbuf, colp.at[pl.ds(base, C)])
            pltpu.sync_copy(rbuf, rowc.at[pl.ds(base, C)])
            return 0

        @pl.when(w < NWS)
        def _():
            lax.fori_loop(0, niter, body, 0)

        plsc.subcore_barrier()
        pltpu.sync_copy(dacc.at[pl.ds(s * rpt, rpt)],
                        degp.at[c, pl.ds(s * rpt, rpt)])

        iota16 = lax.iota(jnp.int32, 16)

        def rep_chunk(drep):
            def one(q, _):
                nb = s * rpt + q * RC
                pltpu.sync_copy(dacc.at[pl.ds(nb, RC)], dbuf)

                def grp(k, _):
                    v = dbuf[pl.ds(k * 16, 16)]
                    for a in range(16):
                        idx = iota16 * F + (k * 16 * F + a)
                        plsc.store_scatter(repbuf, [idx], v)
                    return 0

                lax.fori_loop(0, RC // 16, grp, 0)
                pltpu.sync_copy(repbuf, drep.at[pl.ds(nb * F, RC * F)])
                return 0

            lax.fori_loop(0, rpt // RC, one, 0)

        @pl.when(c == 0)
        def _():
            rep_chunk(d0rep)

        @pl.when(c == 1)
        def _():
            rep_chunk(d1rep)

    return setup


def _sc_layer(E, n, N_PAD, C=800):
    ept = E // NS
    niter = ept // C
    nblk = niter // 6
    head = min(6, niter)
    n_acc = ((n + 1 + 127) // 128) * 128
    rpt = n_acc // NS
    mesh = plsc.VectorSubcoreMesh(core_axis_name="c", subcore_axis_name="s")

    @functools.partial(
        pl.kernel,
        out_type=(
            jax.ShapeDtypeStruct((N_PAD, F), jnp.float32),
            jax.ShapeDtypeStruct((N_PAD, F), jnp.float32),
        ),
        mesh=mesh,
        scratch_types=[
            pltpu.VMEM((C,), jnp.int32),
            pltpu.VMEM((C,), jnp.int32),
            pltpu.VMEM((C,), jnp.int32),
            pltpu.VMEM((C,), jnp.int32),
            pltpu.VMEM((C,), jnp.int32),
            pltpu.VMEM((C,), jnp.int32),
            pltpu.VMEM((C, F), jnp.float32),
            pltpu.VMEM((C, F), jnp.float32),
            pltpu.VMEM_SHARED((n_acc, F), jnp.float32),
            pltpu.SemaphoreType.DMA,
            pltpu.SemaphoreType.DMA,
            pltpu.SemaphoreType.DMA,
            pltpu.SemaphoreType.DMA,
            pltpu.SemaphoreType.DMA,
            pltpu.SemaphoreType.DMA,
            pltpu.SemaphoreType.DMA,
        ],
        compiler_params=pltpu.CompilerParams(use_tc_tiling_on_sc=False),
    )
    def layer(row, colp, ulo, uhi, alo, ahi,
              rb0, cb0, rb1, cb1, rb2, cb2, gb0, gb1, acc,
              si0, si1, si2, sg0, sg1, ss0, ss1):
        c = lax.axis_index("c")
        s = lax.axis_index("s")
        rb, cb, si = (rb0, rb1, rb2), (cb0, cb1, cb2), (si0, si1, si2)
        gb, sg, ss = (gb0, gb1), (sg0, sg1), (ss0, ss1)

        def zrow(i, _):
            gb0[i, :] = jnp.zeros((F,), jnp.float32)
            return 0

        lax.fori_loop(0, C, zrow, 0)
        base = s * rpt
        done = 0
        while done < rpt:
            step = min(C, rpt - done)
            pltpu.sync_copy(gb0.at[pl.ds(0, step)],
                            acc.at[pl.ds(base + done, step)])
            done += step
        plsc.subcore_barrier()

        def run(u_hbm):
            tb = s * ept

            def issue_idx(k, j):
                pltpu.async_copy(row.at[pl.ds(tb + k * C, C)], rb[j], si[j])
                pltpu.async_copy(colp.at[pl.ds(tb + k * C, C)], cb[j], si[j])

            def wait_idx(j):
                pltpu.make_async_copy(row.at[pl.ds(tb, C)], rb[j],
                                      si[j]).wait()
                pltpu.make_async_copy(colp.at[pl.ds(tb, C)], cb[j],
                                      si[j]).wait()

            def issue_gather(j, g):
                pltpu.async_copy(u_hbm.at[rb[j]], gb[g], sg[g])

            def wait_gather(j, g):
                pltpu.make_async_copy(u_hbm.at[rb[j]], gb[g], sg[g]).wait()

            def issue_scat(j, g):
                pltpu.async_copy(gb[g], acc.at[cb[j]], ss[g], add=True)

            def wait_scat(j, g):
                pltpu.make_async_copy(gb[g], acc.at[cb[j]], ss[g]).wait()

            def steps(k0, ks, static):
                for t in range(ks):
                    j, jn = t % 3, (t + 1) % 3
                    g, gn = t % 2, (t + 1) % 2
                    k = k0 + t
                    wait_gather(j, g)
                    issue_scat(j, g)
                    if (not static) or k + 1 < niter:
                        wait_idx(jn)
                    if (not static) or k > 0:
                        wait_scat((t + 2) % 3, gn)
                    if (not static) or k + 1 < niter:
                        issue_gather(jn, gn)
                    if (not static) or k + 2 < niter:
                        issue_idx(k + 2, (t + 2) % 3)

            issue_idx(0, 0)
            issue_idx(1, 1)
            wait_idx(0)
            issue_gather(0, 0)
            steps(0, head, True)

            if nblk > 1:
                def body(b, _):
                    steps(6 * b, 6, False)
                    return 0

                lax.fori_loop(1, nblk, body, 0)
            for k in range(6 * nblk, niter):
                t = k % 6
                j, jn = t % 3, (t + 1) % 3
                g, gn = t % 2, (t + 1) % 2
                wait_gather(j, g)
                issue_scat(j, g)
                if k + 1 < niter:
                    wait_idx(jn)
                wait_scat((t + 2) % 3, gn)
                if k + 1 < niter:
                    issue_gather(jn, gn)
                if k + 2 < niter:
                    issue_idx(k + 2, (t + 2) % 3)
            lt = (niter - 1) % 6
            wait_scat(lt % 3, lt % 2)

        @pl.when(c == 0)
        def _():
            run(ulo)

        @pl.when(c == 1)
        def _():
            run(uhi)

        plsc.subcore_barrier()

        @pl.when(c == 0)
        def _():
            pltpu.sync_copy(acc.at[pl.ds(s * rpt, rpt)],
                            alo.at[pl.ds(s * rpt, rpt)])

        @pl.when(c == 1)
        def _():
            pltpu.sync_copy(acc.at[pl.ds(s * rpt, rpt)],
                            ahi.at[pl.ds(s * rpt, rpt)])

    return layer


def _dis_of(degp_blk):
    deg = degp_blk[0, :] + degp_blk[1, :] + 2.0
    return lax.rsqrt(deg)[:, None]


def _row_mask(n):
    rows = pl.program_id(0) * BN + lax.broadcasted_iota(jnp.int32, (BN, 1), 0)
    return rows < n


def _enc1_body(n, x_ref, w_ref, b_ref, u_ref, st_ref):
    u = jnp.dot(x_ref[...], w_ref[...], precision=_HIGH,
                preferred_element_type=jnp.float32) + b_ref[...]
    u_ref[...] = u
    um = jnp.where(_row_mask(n), u, 0.0)
    st = jnp.stack([jnp.sum(um, axis=0), jnp.sum(um * um, axis=0)])

    @pl.when(pl.program_id(0) == 0)
    def _():
        st_ref[...] = st

    @pl.when(pl.program_id(0) > 0)
    def _():
        st_ref[...] += st


def _bn_elu(u, st, g, be, n):
    mean = st[0:1, :] / n
    var = st[1:2, :] / n - mean * mean
    h = (u - mean) * lax.rsqrt(var + 0.001) * g + be
    return jnp.where(h > 0, h, jnp.exp(h) - 1.0)


def _enc2_body(n, u_ref, st_ref, g_ref, be_ref, w_ref, b_ref, d0_ref, d1_ref,
               v_ref, st2_ref, dis_ref):
    h = _bn_elu(u_ref[...], st_ref[...], g_ref[...], be_ref[...], n)
    v = jnp.dot(h, w_ref[...], precision=_HIGH,
                preferred_element_type=jnp.float32) + b_ref[...]
    v_ref[...] = v
    dis_ref[...] = lax.rsqrt(d0_ref[...] + d1_ref[...] + 2.0)
    vm = jnp.where(_row_mask(n), v, 0.0)
    st = jnp.stack([jnp.sum(vm, axis=0), jnp.sum(vm * vm, axis=0)])

    @pl.when(pl.program_id(0) == 0)
    def _():
        st2_ref[...] = st

    @pl.when(pl.program_id(0) > 0)
    def _():
        st2_ref[...] += st


def _mm1_body(n, v_ref, st_ref, g_ref, be_ref, degp_ref, w_ref,
              ulo_ref, uhi_ref):
    h = _bn_elu(v_ref[...], st_ref[...], g_ref[...], be_ref[...], n)
    t = jnp.dot(h, w_ref[...], precision=_HIGH,
                preferred_element_type=jnp.float32)
    u = _dis_of(degp_ref[...]) * t
    ulo_ref[...] = u[:, :F]
    uhi_ref[...] = u[:, F:]


def _mid_body(relu, alo_ref, ahi_ref, ulo_ref, uhi_ref, dis_ref,
              bl_ref, bh_ref, kll_ref, khl_ref, klh_ref, khh_ref,
              olo_ref, ohi_ref):
    dis = dis_ref[...]
    zl = dis * alo_ref[...] + 2.0 * dis * ulo_ref[...] + bl_ref[...]
    zh = dis * ahi_ref[...] + 2.0 * dis * uhi_ref[...] + bh_ref[...]
    if relu:
        zl = jnp.maximum(zl, 0.0)
        zh = jnp.maximum(zh, 0.0)
    dot = functools.partial(jnp.dot, precision=_HIGH,
                            preferred_element_type=jnp.float32)
    olo_ref[...] = dis * (dot(zl, kll_ref[...]) + dot(zh, khl_ref[...]))
    ohi_ref[...] = dis * (dot(zl, klh_ref[...]) + dot(zh, khh_ref[...]))


def _fin_body(alo_ref, ahi_ref, ulo_ref, uhi_ref, dis_ref, bl_ref, bh_ref,
              olo_ref, ohi_ref):
    dis = dis_ref[...]
    olo_ref[...] = dis * alo_ref[...] + 2.0 * dis * ulo_ref[...] + bl_ref[...]
    ohi_ref[...] = dis * ahi_ref[...] + 2.0 * dis * uhi_ref[...] + bh_ref[...]


def _row_spec(cols):
    return pl.BlockSpec((BN, cols), lambda i: (i, 0))


def _full_spec(shape):
    return pl.BlockSpec(shape, lambda i: tuple(0 for _ in shape))


def _degp_spec():
    return pl.BlockSpec((NC, BN), lambda i: (0, i))


def kernel(x, edge_index, W1, b1, g1, be1, W2, b2, g2, be2,
           Wg1, bg1, Wg2, bg2, Wg3, bg3):
    n, d = x.shape
    e = edge_index.shape[1]
    n_pad = _npad(n)
    d1 = W1.shape[1]
    d2 = W2.shape[1]
    grid = (n_pad // BN,)

    m8 = n_pad // 8
    mb = BN // 8
    b1r, g1r, be1r = b1[None, :], g1[None, :], be1[None, :]
    b2r, g2r, be2r = b2[None, :], g2[None, :], be2[None, :]
    i8 = jnp.eye(8, dtype=jnp.float32)

    def krons(w):
        return (jnp.kron(i8, w[:F, :F]), jnp.kron(i8, w[F:, :F]),
                jnp.kron(i8, w[:F, F:]), jnp.kron(i8, w[F:, F:]))

    def btiles(b):
        return jnp.tile(b[:F], 8)[None, :], jnp.tile(b[F:], 8)[None, :]

    degp, colp, row, d0rep, d1rep = _sc_setup(e, n_pad, n)(edge_index)
    d0p = jnp.reshape(d0rep, (m8, 128))
    d1p = jnp.reshape(d1rep, (m8, 128))

    U, st1 = pl.pallas_call(
        functools.partial(_enc1_body, float(n)),
        grid=grid,
        in_specs=[_row_spec(d), _full_spec((d, d1)), _full_spec((1, d1))],
        out_specs=[_row_spec(d1), _full_spec((2, d1))],
        out_shape=[jax.ShapeDtypeStruct((n_pad, d1), jnp.float32),
                   jax.ShapeDtypeStruct((2, d1), jnp.float32)],
        compiler_params=pltpu.CompilerParams(
            dimension_semantics=("arbitrary",)),
    )(x, W1, b1r)

    V, st2, dis_p = pl.pallas_call(
        functools.partial(_enc2_body, float(n)),
        grid=grid,
        in_specs=[_row_spec(d1), _full_spec((2, d1)),
                  _full_spec((1, d1)), _full_spec((1, d1)),
                  _full_spec((d1, d2)), _full_spec((1, d2)),
                  pl.BlockSpec((mb, 128), lambda i: (i, 0)),
                  pl.BlockSpec((mb, 128), lambda i: (i, 0))],
        out_specs=[_row_spec(d2), _full_spec((2, d2)),
                   pl.BlockSpec((mb, 128), lambda i: (i, 0))],
        out_shape=[jax.ShapeDtypeStruct((n_pad, d2), jnp.float32),
                   jax.ShapeDtypeStruct((2, d2), jnp.float32),
                   jax.ShapeDtypeStruct((m8, 128), jnp.float32)],
        compiler_params=pltpu.CompilerParams(
            dimension_semantics=("arbitrary",)),
    )(U, st1, g1r, be1r, W2, b2r, d0p, d1p)

    ulo, uhi = pl.pallas_call(
        functools.partial(_mm1_body, float(n)),
        grid=grid,
        in_specs=[_row_spec(d2), _full_spec((2, d2)),
                  _full_spec((1, d2)), _full_spec((1, d2)),
                  _degp_spec(), _full_spec((d2, d2))],
        out_specs=[_row_spec(F), _row_spec(F)],
        out_shape=[jax.ShapeDtypeStruct((n_pad, F), jnp.float32),
                   jax.ShapeDtypeStruct((n_pad, F), jnp.float32)],
        compiler_params=pltpu.CompilerParams(
            dimension_semantics=("parallel",)),
    )(V, st2, g2r, be2r, degp, Wg1)

    sc_layer = _sc_layer(e, n, n_pad)
    pspec = pl.BlockSpec((mb, 128), lambda i: (i, 0))

    def mid(relu, alo_p, ahi_p, ulo_p, uhi_p, bprev, wg):
        kll, khl, klh, khh = krons(wg)
        bl, bh = btiles(bprev)
        return pl.pallas_call(
            functools.partial(_mid_body, relu),
            grid=grid,
            in_specs=[pspec, pspec, pspec, pspec, pspec,
                      _full_spec((1, 128)), _full_spec((1, 128)),
                      _full_spec((128, 128)), _full_spec((128, 128)),
                      _full_spec((128, 128)), _full_spec((128, 128))],
            out_specs=[pspec, pspec],
            out_shape=[jax.ShapeDtypeStruct((m8, 128), jnp.float32),
                       jax.ShapeDtypeStruct((m8, 128), jnp.float32)],
            compiler_params=pltpu.CompilerParams(
                dimension_semantics=("parallel",)),
        )(alo_p, ahi_p, ulo_p, uhi_p, dis_p, bl, bh, kll, khl, klh, khh)

    def as_pack(a_lin):
        return jnp.reshape(a_lin, (m8, 128))

    def as_lin(a_p):
        return jnp.reshape(a_p, (n_pad, F))

    u1lo_p = lax.optimization_barrier(as_pack(ulo))
    u1hi_p = lax.optimization_barrier(as_pack(uhi))

    alo1, ahi1 = sc_layer(row, colp, as_lin(u1lo_p), as_lin(u1hi_p))
    ulo2_p, uhi2_p = mid(True, as_pack(alo1), as_pack(ahi1),
                         u1lo_p, u1hi_p, bg1, Wg2)
    alo2, ahi2 = sc_layer(row, colp, as_lin(ulo2_p), as_lin(uhi2_p))
    ulo3_p, uhi3_p = mid(False, as_pack(alo2), as_pack(ahi2),
                         ulo2_p, uhi2_p, bg2, Wg3)
    alo3, ahi3 = sc_layer(row, colp, as_lin(ulo3_p), as_lin(uhi3_p))

    bl3, bh3 = btiles(bg3)
    zlo_p, zhi_p = pl.pallas_call(
        _fin_body,
        grid=grid,
        in_specs=[pspec, pspec, pspec, pspec, pspec,
                  _full_spec((1, 128)), _full_spec((1, 128))],
        out_specs=[pspec, pspec],
        out_shape=[jax.ShapeDtypeStruct((m8, 128), jnp.float32),
                   jax.ShapeDtypeStruct((m8, 128), jnp.float32)],
        compiler_params=pltpu.CompilerParams(
            dimension_semantics=("parallel",)),
    )(as_pack(alo3), as_pack(ahi3), ulo3_p, uhi3_p, dis_p, bl3, bh3)

    h3 = jnp.concatenate([jnp.reshape(zlo_p, (n_pad, F)),
                          jnp.reshape(zhi_p, (n_pad, F))], axis=1)
    return h3[:n]


# speedup vs baseline: 45.2627x; 1.0055x over previous
import functools

import jax
import jax.numpy as jnp
from jax import lax
from jax.experimental import pallas as pl
from jax.experimental.pallas import tpu as pltpu
from jax.experimental.pallas import tpu_sc as plsc

NC = 2
NS = 16
F = 16
BN = 4096

_HIGH = lax.Precision.DEFAULT


def _npad(n):
    return ((n + 1 + BN - 1) // BN) * BN


def _sc_setup(E, N_PAD, trash, C=2560, NWS=25):
    epw = E // NWS
    niter = epw // C
    rpt = N_PAD // NS
    mesh = plsc.VectorSubcoreMesh(core_axis_name="c", subcore_axis_name="s")

    RC = 800

    @functools.partial(
        pl.kernel,
        out_type=(
            jax.ShapeDtypeStruct((NC, N_PAD), jnp.float32),
            jax.ShapeDtypeStruct((E,), jnp.int32),
            jax.ShapeDtypeStruct((E,), jnp.int32),
            jax.ShapeDtypeStruct((N_PAD * F,), jnp.float32),
            jax.ShapeDtypeStruct((N_PAD * F,), jnp.float32),
        ),
        mesh=mesh,
        scratch_types=[
            pltpu.VMEM((2, C), jnp.int32),
            pltpu.VMEM((C,), jnp.int32),
            pltpu.VMEM((C,), jnp.float32),
            pltpu.VMEM((C,), jnp.int32),
            pltpu.VMEM((rpt,), jnp.float32),
            pltpu.VMEM((RC,), jnp.float32),
            pltpu.VMEM((RC * F,), jnp.float32),
            pltpu.VMEM_SHARED((N_PAD,), jnp.float32),
        ],
        compiler_params=pltpu.CompilerParams(needs_layout_passes=False),
    )
    def setup(ei, degp, colp, rowc, d0rep, d1rep,
              ebuf, rbuf, wbuf, cpbuf, zbuf, dbuf, repbuf, dacc):
        c = lax.axis_index("c")
        s = lax.axis_index("s")
        w = s * NC + c

        def zrow(i, _):
            zbuf[pl.ds(i * 16, 16)] = jnp.zeros((16,), jnp.float32)
            return 0

        lax.fori_loop(0, rpt // 16, zrow, 0)
        pltpu.sync_copy(zbuf, dacc.at[pl.ds(s * rpt, rpt)])
        plsc.subcore_barrier()

        def body(i, _):
            base = w * epw + i * C
            pltpu.sync_copy(ei.at[:, pl.ds(base, C)], ebuf)

            def vec(k, _):
                sl = pl.ds(k * 16, 16)
                r = ebuf[0, sl]
                cc = ebuf[1, sl]
                m = r == cc
                rbuf[sl] = r
                wbuf[sl] = jnp.where(m, 0.0, 1.0).astype(jnp.float32)
                cpbuf[sl] = jnp.where(m, trash, cc)
                return 0

            lax.fori_loop(0, C // 16, vec, 0)
            pltpu.sync_copy(wbuf, dacc.at[rbuf], add=True)
            pltpu.sync_copy(cpbuf, colp.at[pl.ds(base, C)])
            pltpu.sync_copy(rbuf, rowc.at[pl.ds(base, C)])
            return 0

        @pl.when(w < NWS)
        def _():
            lax.fori_loop(0, niter, body, 0)

        plsc.subcore_barrier()
        pltpu.sync_copy(dacc.at[pl.ds(s * rpt, rpt)],
                        degp.at[c, pl.ds(s * rpt, rpt)])

        iota16 = lax.iota(jnp.int32, 16)

        def rep_chunk(drep):
            def one(q, _):
                nb = s * rpt + q * RC
                pltpu.sync_copy(dacc.at[pl.ds(nb, RC)], dbuf)

                def grp(k, _):
                    v = dbuf[pl.ds(k * 16, 16)]
                    for a in range(16):
                        idx = iota16 * F + (k * 16 * F + a)
                        plsc.store_scatter(repbuf, [idx], v)
                    return 0

                lax.fori_loop(0, RC // 16, grp, 0)
                pltpu.sync_copy(repbuf, drep.at[pl.ds(nb * F, RC * F)])
                return 0

            lax.fori_loop(0, rpt // RC, one, 0)

        @pl.when(c == 0)
        def _():
            rep_chunk(d0rep)

        @pl.when(c == 1)
        def _():
            rep_chunk(d1rep)

    return setup


def _sc_layer(E, n, N_PAD, C=800):
    ept = E // NS
    niter = ept // C
    nblk = niter // 6
    head = min(6, niter)
    n_acc = ((n + 1 + 127) // 128) * 128
    rpt = n_acc // NS
    mesh = plsc.VectorSubcoreMesh(core_axis_name="c", subcore_axis_name="s")

    @functools.partial(
        pl.kernel,
        out_type=(
            jax.ShapeDtypeStruct((N_PAD, F), jnp.float32),
            jax.ShapeDtypeStruct((N_PAD, F), jnp.float32),
        ),
        mesh=mesh,
        scratch_types=[
            pltpu.VMEM((C,), jnp.int32),
            pltpu.VMEM((C,), jnp.int32),
            pltpu.VMEM((C,), jnp.int32),
            pltpu.VMEM((C,), jnp.int32),
            pltpu.VMEM((C,), jnp.int32),
            pltpu.VMEM((C,), jnp.int32),
            pltpu.VMEM((C, F), jnp.float32),
            pltpu.VMEM((C, F), jnp.float32),
            pltpu.VMEM_SHARED((n_acc, F), jnp.float32),
            pltpu.SemaphoreType.DMA,
            pltpu.SemaphoreType.DMA,
            pltpu.SemaphoreType.DMA,
            pltpu.SemaphoreType.DMA,
            pltpu.SemaphoreType.DMA,
            pltpu.SemaphoreType.DMA,
            pltpu.SemaphoreType.DMA,
        ],
        compiler_params=pltpu.CompilerParams(use_tc_tiling_on_sc=False),
    )
    def layer(row, colp, ulo, uhi, alo, ahi,
              rb0, cb0, rb1, cb1, rb2, cb2, gb0, gb1, acc,
              si0, si1, si2, sg0, sg1, ss0, ss1):
        c = lax.axis_index("c")
        s = lax.axis_index("s")
        rb, cb, si = (rb0, rb1, rb2), (cb0, cb1, cb2), (si0, si1, si2)
        gb, sg, ss = (gb0, gb1), (sg0, sg1), (ss0, ss1)

        def run(u_hbm):
            tb = s * ept

            def issue_idx(k, j):
                pltpu.async_copy(row.at[pl.ds(tb + k * C, C)], rb[j], si[j])
                pltpu.async_copy(colp.at[pl.ds(tb + k * C, C)], cb[j], si[j])

            def wait_idx(j):
                pltpu.make_async_copy(row.at[pl.ds(tb, C)], rb[j],
                                      si[j]).wait()
                pltpu.make_async_copy(colp.at[pl.ds(tb, C)], cb[j],
                                      si[j]).wait()

            def issue_gather(j, g):
                pltpu.async_copy(u_hbm.at[rb[j]], gb[g], sg[g])

            def wait_gather(j, g):
                pltpu.make_async_copy(u_hbm.at[rb[j]], gb[g], sg[g]).wait()

            def issue_scat(j, g):
                pltpu.async_copy(gb[g], acc.at[cb[j]], ss[g], add=True)

            def wait_scat(j, g):
                pltpu.make_async_copy(gb[g], acc.at[cb[j]], ss[g]).wait()

            def steps(k0, ks, static):
                for t in range(ks):
                    j, jn = t % 3, (t + 1) % 3
                    g, gn = t % 2, (t + 1) % 2
                    k = k0 + t
                    wait_gather(j, g)
                    issue_scat(j, g)
                    if (not static) or k + 1 < niter:
                        wait_idx(jn)
                    if (not static) or k > 0:
                        wait_scat((t + 2) % 3, gn)
                    if (not static) or k + 1 < niter:
                        issue_gather(jn, gn)
                    if (not static) or k + 2 < niter:
                        issue_idx(k + 2, (t + 2) % 3)

            issue_idx(0, 0)
            issue_idx(1, 1)

            def zrow(i, _):
                gb1[i, :] = jnp.zeros((F,), jnp.float32)
                return 0

            lax.fori_loop(0, C, zrow, 0)
            wait_idx(0)
            issue_gather(0, 0)
            base = s * rpt
            done = 0
            while done < rpt:
                step = min(C, rpt - done)
                pltpu.sync_copy(gb1.at[pl.ds(0, step)],
                                acc.at[pl.ds(base + done, step)])
                done += step
            plsc.subcore_barrier()
            steps(0, head, True)

            if nblk > 1:
                def body(b, _):
                    steps(6 * b, 6, False)
                    return 0

                lax.fori_loop(1, nblk, body, 0)
            for k in range(6 * nblk, niter):
                t = k % 6
                j, jn = t % 3, (t + 1) % 3
                g, gn = t % 2, (t + 1) % 2
                wait_gather(j, g)
                issue_scat(j, g)
                if k + 1 < niter:
                    wait_idx(jn)
                wait_scat((t + 2) % 3, gn)
                if k + 1 < niter:
                    issue_gather(jn, gn)
                if k + 2 < niter:
                    issue_idx(k + 2, (t + 2) % 3)
            lt = (niter - 1) % 6
            wait_scat(lt % 3, lt % 2)

        @pl.when(c == 0)
        def _():
            run(ulo)

        @pl.when(c == 1)
        def _():
            run(uhi)

        plsc.subcore_barrier()

        @pl.when(c == 0)
        def _():
            pltpu.sync_copy(acc.at[pl.ds(s * rpt, rpt)],
                            alo.at[pl.ds(s * rpt, rpt)])

        @pl.when(c == 1)
        def _():
            pltpu.sync_copy(acc.at[pl.ds(s * rpt, rpt)],
                            ahi.at[pl.ds(s * rpt, rpt)])

    return layer


def _dis_of(degp_blk):
    deg = degp_blk[0, :] + degp_blk[1, :] + 2.0
    return lax.rsqrt(deg)[:, None]


def _row_mask(n):
    rows = pl.program_id(0) * BN + lax.broadcasted_iota(jnp.int32, (BN, 1), 0)
    return rows < n


def _enc1_body(n, x_ref, w_ref, b_ref, u_ref, st_ref):
    u = jnp.dot(x_ref[...], w_ref[...], precision=_HIGH,
                preferred_element_type=jnp.float32) + b_ref[...]
    u_ref[...] = u
    um = jnp.where(_row_mask(n), u, 0.0)
    st = jnp.stack([jnp.sum(um, axis=0), jnp.sum(um * um, axis=0)])

    @pl.when(pl.program_id(0) == 0)
    def _():
        st_ref[...] = st

    @pl.when(pl.program_id(0) > 0)
    def _():
        st_ref[...] += st


def _bn_elu(u, st, g, be, n):
    mean = st[0:1, :] / n
    var = st[1:2, :] / n - mean * mean
    h = (u - mean) * lax.rsqrt(var + 0.001) * g + be
    return jnp.where(h > 0, h, jnp.exp(h) - 1.0)


def _enc2_body(n, u_ref, st_ref, g_ref, be_ref, w_ref, b_ref, d0_ref, d1_ref,
               v_ref, st2_ref, dis_ref):
    h = _bn_elu(u_ref[...], st_ref[...], g_ref[...], be_ref[...], n)
    v = jnp.dot(h, w_ref[...], precision=_HIGH,
                preferred_element_type=jnp.float32) + b_ref[...]
    v_ref[...] = v
    dis_ref[...] = lax.rsqrt(d0_ref[...] + d1_ref[...] + 2.0)
    vm = jnp.where(_row_mask(n), v, 0.0)
    st = jnp.stack([jnp.sum(vm, axis=0), jnp.sum(vm * vm, axis=0)])

    @pl.when(pl.program_id(0) == 0)
    def _():
        st2_ref[...] = st

    @pl.when(pl.program_id(0) > 0)
    def _():
        st2_ref[...] += st


def _mm1_body(n, v_ref, st_ref, g_ref, be_ref, degp_ref, w_ref,
              ulo_ref, uhi_ref):
    h = _bn_elu(v_ref[...], st_ref[...], g_ref[...], be_ref[...], n)
    t = jnp.dot(h, w_ref[...], precision=_HIGH,
                preferred_element_type=jnp.float32)
    u = _dis_of(degp_ref[...]) * t
    ulo_ref[...] = u[:, :F]
    uhi_ref[...] = u[:, F:]


def _mid_body(relu, alo_ref, ahi_ref, ulo_ref, uhi_ref, dis_ref,
              bl_ref, bh_ref, kll_ref, khl_ref, klh_ref, khh_ref,
              olo_ref, ohi_ref):
    dis = dis_ref[...]
    zl = dis * alo_ref[...] + 2.0 * dis * ulo_ref[...] + bl_ref[...]
    zh = dis * ahi_ref[...] + 2.0 * dis * uhi_ref[...] + bh_ref[...]
    if relu:
        zl = jnp.maximum(zl, 0.0)
        zh = jnp.maximum(zh, 0.0)
    dot = functools.partial(jnp.dot, precision=_HIGH,
                            preferred_element_type=jnp.float32)
    olo_ref[...] = dis * (dot(zl, kll_ref[...]) + dot(zh, khl_ref[...]))
    ohi_ref[...] = dis * (dot(zl, klh_ref[...]) + dot(zh, khh_ref[...]))


def _fin_body(alo_ref, ahi_ref, ulo_ref, uhi_ref, dis_ref, bl_ref, bh_ref,
              olo_ref, ohi_ref):
    dis = dis_ref[...]
    olo_ref[...] = dis * alo_ref[...] + 2.0 * dis * ulo_ref[...] + bl_ref[...]
    ohi_ref[...] = dis * ahi_ref[...] + 2.0 * dis * uhi_ref[...] + bh_ref[...]


def _row_spec(cols):
    return pl.BlockSpec((BN, cols), lambda i: (i, 0))


def _full_spec(shape):
    return pl.BlockSpec(shape, lambda i: tuple(0 for _ in shape))


def _degp_spec():
    return pl.BlockSpec((NC, BN), lambda i: (0, i))


def kernel(x, edge_index, W1, b1, g1, be1, W2, b2, g2, be2,
           Wg1, bg1, Wg2, bg2, Wg3, bg3):
    n, d = x.shape
    e = edge_index.shape[1]
    n_pad = _npad(n)
    d1 = W1.shape[1]
    d2 = W2.shape[1]
    grid = (n_pad // BN,)

    m8 = n_pad // 8
    mb = BN // 8
    b1r, g1r, be1r = b1[None, :], g1[None, :], be1[None, :]
    b2r, g2r, be2r = b2[None, :], g2[None, :], be2[None, :]
    i8 = jnp.eye(8, dtype=jnp.float32)

    def krons(w):
        return (jnp.kron(i8, w[:F, :F]), jnp.kron(i8, w[F:, :F]),
                jnp.kron(i8, w[:F, F:]), jnp.kron(i8, w[F:, F:]))

    def btiles(b):
        return jnp.tile(b[:F], 8)[None, :], jnp.tile(b[F:], 8)[None, :]

    degp, colp, row, d0rep, d1rep = _sc_setup(e, n_pad, n)(edge_index)
    d0p = jnp.reshape(d0rep, (m8, 128))
    d1p = jnp.reshape(d1rep, (m8, 128))

    U, st1 = pl.pallas_call(
        functools.partial(_enc1_body, float(n)),
        grid=grid,
        in_specs=[_row_spec(d), _full_spec((d, d1)), _full_spec((1, d1))],
        out_specs=[_row_spec(d1), _full_spec((2, d1))],
        out_shape=[jax.ShapeDtypeStruct((n_pad, d1), jnp.float32),
                   jax.ShapeDtypeStruct((2, d1), jnp.float32)],
        compiler_params=pltpu.CompilerParams(
            dimension_semantics=("arbitrary",)),
    )(x, W1, b1r)

    V, st2, dis_p = pl.pallas_call(
        functools.partial(_enc2_body, float(n)),
        grid=grid,
        in_specs=[_row_spec(d1), _full_spec((2, d1)),
                  _full_spec((1, d1)), _full_spec((1, d1)),
                  _full_spec((d1, d2)), _full_spec((1, d2)),
                  pl.BlockSpec((mb, 128), lambda i: (i, 0)),
                  pl.BlockSpec((mb, 128), lambda i: (i, 0))],
        out_specs=[_row_spec(d2), _full_spec((2, d2)),
                   pl.BlockSpec((mb, 128), lambda i: (i, 0))],
        out_shape=[jax.ShapeDtypeStruct((n_pad, d2), jnp.float32),
                   jax.ShapeDtypeStruct((2, d2), jnp.float32),
                   jax.ShapeDtypeStruct((m8, 128), jnp.float32)],
        compiler_params=pltpu.CompilerParams(
            dimension_semantics=("arbitrary",)),
    )(U, st1, g1r, be1r, W2, b2r, d0p, d1p)

    ulo, uhi = pl.pallas_call(
        functools.partial(_mm1_body, float(n)),
        grid=grid,
        in_specs=[_row_spec(d2), _full_spec((2, d2)),
                  _full_spec((1, d2)), _full_spec((1, d2)),
                  _degp_spec(), _full_spec((d2, d2))],
        out_specs=[_row_spec(F), _row_spec(F)],
        out_shape=[jax.ShapeDtypeStruct((n_pad, F), jnp.float32),
                   jax.ShapeDtypeStruct((n_pad, F), jnp.float32)],
        compiler_params=pltpu.CompilerParams(
            dimension_semantics=("parallel",)),
    )(V, st2, g2r, be2r, degp, Wg1)

    sc_layer = _sc_layer(e, n, n_pad)
    pspec = pl.BlockSpec((mb, 128), lambda i: (i, 0))

    def mid(relu, alo_p, ahi_p, ulo_p, uhi_p, bprev, wg):
        kll, khl, klh, khh = krons(wg)
        bl, bh = btiles(bprev)
        return pl.pallas_call(
            functools.partial(_mid_body, relu),
            grid=grid,
            in_specs=[pspec, pspec, pspec, pspec, pspec,
                      _full_spec((1, 128)), _full_spec((1, 128)),
                      _full_spec((128, 128)), _full_spec((128, 128)),
                      _full_spec((128, 128)), _full_spec((128, 128))],
            out_specs=[pspec, pspec],
            out_shape=[jax.ShapeDtypeStruct((m8, 128), jnp.float32),
                       jax.ShapeDtypeStruct((m8, 128), jnp.float32)],
            compiler_params=pltpu.CompilerParams(
                dimension_semantics=("parallel",)),
        )(alo_p, ahi_p, ulo_p, uhi_p, dis_p, bl, bh, kll, khl, klh, khh)

    def as_pack(a_lin):
        return jnp.reshape(a_lin, (m8, 128))

    def as_lin(a_p):
        return jnp.reshape(a_p, (n_pad, F))

    u1lo_p = lax.optimization_barrier(as_pack(ulo))
    u1hi_p = lax.optimization_barrier(as_pack(uhi))

    alo1, ahi1 = sc_layer(row, colp, as_lin(u1lo_p), as_lin(u1hi_p))
    ulo2_p, uhi2_p = mid(True, as_pack(alo1), as_pack(ahi1),
                         u1lo_p, u1hi_p, bg1, Wg2)
    alo2, ahi2 = sc_layer(row, colp, as_lin(ulo2_p), as_lin(uhi2_p))
    ulo3_p, uhi3_p = mid(False, as_pack(alo2), as_pack(ahi2),
                         ulo2_p, uhi2_p, bg2, Wg3)
    alo3, ahi3 = sc_layer(row, colp, as_lin(ulo3_p), as_lin(uhi3_p))

    bl3, bh3 = btiles(bg3)
    zlo_p, zhi_p = pl.pallas_call(
        _fin_body,
        grid=grid,
        in_specs=[pspec, pspec, pspec, pspec, pspec,
                  _full_spec((1, 128)), _full_spec((1, 128))],
        out_specs=[pspec, pspec],
        out_shape=[jax.ShapeDtypeStruct((m8, 128), jnp.float32),
                   jax.ShapeDtypeStruct((m8, 128), jnp.float32)],
        compiler_params=pltpu.CompilerParams(
            dimension_semantics=("parallel",)),
    )(as_pack(alo3), as_pack(ahi3), ulo3_p, uhi3_p, dis_p, bl3, bh3)

    h3 = jnp.concatenate([jnp.reshape(zlo_p, (n_pad, F)),
                          jnp.reshape(zhi_p, (n_pad, F))], axis=1)
    return h3[:n]
